# Initial kernel scaffold; baseline (speedup 1.0000x reference)
import functools

import jax
import jax.numpy as jnp
from jax import lax
from jax.experimental import pallas as pl
from jax.experimental.pallas import tpu as pltpu
from jax.experimental.pallas import tpu_sc as plsc

_NC = 2
_NS = 16
_NW = _NC * _NS
_B = 128
_DEN = 16
_NPAD_SUB = 640


def _mm2_body(x_ref, wl_ref, wr_ref, xl_ref, xr_ref):
    x = x_ref[...]
    xl_ref[...] = jnp.dot(x, wl_ref[...], preferred_element_type=jnp.float32)
    xr_ref[...] = jnp.dot(x, wr_ref[...], preferred_element_type=jnp.float32)


def _tc_mm2(x, wl, wr):
    n, d = x.shape
    h = wl.shape[1]
    nb = 1000
    return pl.pallas_call(
        _mm2_body,
        grid=(n // nb,),
        in_specs=[
            pl.BlockSpec((nb, d), lambda i: (i, 0)),
            pl.BlockSpec((d, h), lambda i: (0, 0)),
            pl.BlockSpec((d, h), lambda i: (0, 0)),
        ],
        out_specs=[
            pl.BlockSpec((nb, h), lambda i: (i, 0)),
            pl.BlockSpec((nb, h), lambda i: (i, 0)),
        ],
        out_shape=[jax.ShapeDtypeStruct((n, h), jnp.float32)] * 2,
    )(x, wl, wr)


def _edge_body(ml_ref, mr_ref, att_ref, w_ref, a_ref, *, e_total, eb):
    i = pl.program_id(0)
    ml = ml_ref[...]
    m = ml + mr_ref[...]
    m = jnp.where(m > 0, m, 0.2 * m)
    e = jnp.sum(m * att_ref[...], axis=1, keepdims=True)
    rows = i * eb + lax.broadcasted_iota(jnp.int32, (eb, 1), 0)
    a = jnp.where(rows < e_total, jnp.exp(e), 0.0)
    w_ref[...] = a * ml
    a_ref[...] = jnp.broadcast_to(a, (eb, _DEN))


def _tc_edge(ml, mr, att, e_total):
    ep, h = ml.shape
    eb = 4096
    body = functools.partial(_edge_body, e_total=e_total, eb=eb)
    return pl.pallas_call(
        body,
        grid=(ep // eb,),
        in_specs=[
            pl.BlockSpec((eb, h), lambda i: (i, 0)),
            pl.BlockSpec((eb, h), lambda i: (i, 0)),
            pl.BlockSpec((1, h), lambda i: (0, 0)),
        ],
        out_specs=[
            pl.BlockSpec((eb, h), lambda i: (i, 0)),
            pl.BlockSpec((eb, _DEN), lambda i: (i, 0)),
        ],
        out_shape=[
            jax.ShapeDtypeStruct((ep, h), jnp.float32),
            jax.ShapeDtypeStruct((ep, _DEN), jnp.float32),
        ],
    )(ml, mr, att.reshape(1, h))


def _finish_body(p1_ref, p2_ref, b_ref, o_ref, *, last):
    acc = p1_ref[0] + p1_ref[1]
    den = p2_ref[0, :, 0:1] + p2_ref[1, :, 0:1]
    out = acc / den + b_ref[...]
    if last:
        mx = jnp.max(out, axis=1, keepdims=True)
        s = out - mx
        out = s - jnp.log(jnp.sum(jnp.exp(s), axis=1, keepdims=True))
    else:
        out = jnp.maximum(out, 0.0)
    o_ref[...] = out


def _tc_finish(p1, p2, b, n, last):
    h = p1.shape[2]
    nb = 1000
    body = functools.partial(_finish_body, last=last)
    return pl.pallas_call(
        body,
        grid=(n // nb,),
        in_specs=[
            pl.BlockSpec((2, nb, h), lambda i: (0, i, 0)),
            pl.BlockSpec((2, nb, _DEN), lambda i: (0, i, 0)),
            pl.BlockSpec((1, h), lambda i: (0, 0)),
        ],
        out_specs=pl.BlockSpec((nb, h), lambda i: (i, 0)),
        out_shape=jax.ShapeDtypeStruct((n, h), jnp.float32),
    )(p1, p2, b.reshape(1, h))


def _sc_gather(xl, xr, srcp, dstp):
    h = xl.shape[1]
    ep = srcp.shape[0]
    per_w = ep // _NW
    chunks = per_w // _B
    mesh = plsc.VectorSubcoreMesh(core_axis_name="c", subcore_axis_name="s")

    @functools.partial(
        pl.kernel,
        out_type=[jax.ShapeDtypeStruct((ep, h), jnp.float32)] * 2,
        mesh=mesh,
        scratch_types=[
            pltpu.VMEM((_B,), jnp.int32),
            pltpu.VMEM((_B, h), jnp.float32),
            pltpu.VMEM((_B,), jnp.int32),
            pltpu.VMEM((_B, h), jnp.float32),
            pltpu.SemaphoreType.DMA,
            pltpu.SemaphoreType.DMA,
        ],
    )
    def k(xl_hbm, xr_hbm, si_hbm, di_hbm, ml_hbm, mr_hbm,
          si_v, rl_v, di_v, rr_v, sem_l, sem_r):
        wid = lax.axis_index("s") * _NC + lax.axis_index("c")

        @pl.loop(0, chunks)
        def _(i):
            base = wid * per_w + i * _B
            pltpu.sync_copy(si_hbm.at[pl.ds(base, _B)], si_v)
            pltpu.sync_copy(di_hbm.at[pl.ds(base, _B)], di_v)
            cl = pltpu.async_copy(xl_hbm.at[si_v], rl_v, sem_l)
            cr = pltpu.async_copy(xr_hbm.at[di_v], rr_v, sem_r)
            cl.wait()
            cr.wait()
            pltpu.sync_copy(rl_v, ml_hbm.at[pl.ds(base, _B)])
            pltpu.sync_copy(rr_v, mr_hbm.at[pl.ds(base, _B)])

    return k(xl, xr, srcp, dstp)


def _sc_scatter(w, a16, dstp, z1, z2):
    ep, h = w.shape
    npad = z1.shape[0]
    per_w = ep // _NW
    chunks = per_w // _B
    mesh = plsc.VectorSubcoreMesh(core_axis_name="c", subcore_axis_name="s")

    @functools.partial(
        pl.kernel,
        out_type=[
            jax.ShapeDtypeStruct((_NC, npad, h), jnp.float32),
            jax.ShapeDtypeStruct((_NC, npad, _DEN), jnp.float32),
        ],
        mesh=mesh,
        scratch_types=[
            pltpu.VMEM((_B,), jnp.int32),
            pltpu.VMEM((_B, h), jnp.float32),
            pltpu.VMEM((_B, _DEN), jnp.float32),
            pltpu.VMEM_SHARED((npad, h), jnp.float32),
            pltpu.VMEM_SHARED((npad, _DEN), jnp.float32),
        ],
    )
    def k(w_hbm, a_hbm, di_hbm, z1_hbm, z2_hbm, o1_hbm, o2_hbm,
          di_v, w_v, a_v, acc1, acc2):
        cid = lax.axis_index("c")
        sid = lax.axis_index("s")
        r0 = sid * _NPAD_SUB
        pltpu.sync_copy(z1_hbm.at[pl.ds(r0, _NPAD_SUB)],
                        acc1.at[pl.ds(r0, _NPAD_SUB)])
        pltpu.sync_copy(z2_hbm.at[pl.ds(r0, _NPAD_SUB)],
                        acc2.at[pl.ds(r0, _NPAD_SUB)])
        plsc.subcore_barrier()
        wid = sid * _NC + cid

        @pl.loop(0, chunks)
        def _(i):
            base = wid * per_w + i * _B
            pltpu.sync_copy(di_hbm.at[pl.ds(base, _B)], di_v)
            pltpu.sync_copy(w_hbm.at[pl.ds(base, _B)], w_v)
            pltpu.sync_copy(a_hbm.at[pl.ds(base, _B)], a_v)
            pltpu.sync_copy(w_v, acc1.at[di_v], add=True)
            pltpu.sync_copy(a_v, acc2.at[di_v], add=True)

        plsc.subcore_barrier()
        pltpu.sync_copy(acc1.at[pl.ds(r0, _NPAD_SUB)],
                        o1_hbm.at[cid, pl.ds(r0, _NPAD_SUB)])
        pltpu.sync_copy(acc2.at[pl.ds(r0, _NPAD_SUB)],
                        o2_hbm.at[cid, pl.ds(r0, _NPAD_SUB)])

    return k(w, a16, dstp, z1, z2)


def kernel(x, edge_index, W1l, W1r, a1, b1, W2l, W2r, a2, b2, W3l, W3r, a3, b3):
    n = x.shape[0]
    e = edge_index.shape[1]
    npad = _NS * _NPAD_SUB
    loops = jnp.arange(n, dtype=edge_index.dtype)
    src = jnp.concatenate([edge_index[0], loops])
    dst = jnp.concatenate([edge_index[1], loops])
    et = e + n
    grain = _NW * _B
    ep = ((et + grain - 1) // grain) * grain
    pad = ep - et
    srcp = jnp.concatenate([src, jnp.zeros((pad,), src.dtype)])
    dstp = jnp.concatenate([dst, jnp.zeros((pad,), dst.dtype)])

    h = x
    for wl, wr, att, b, last in (
        (W1l, W1r, a1, b1, False),
        (W2l, W2r, a2, b2, False),
        (W3l, W3r, a3, b3, True),
    ):
        hdim = wl.shape[1]
        xl, xr = _tc_mm2(h, wl, wr)
        ml, mr = _sc_gather(xl, xr, srcp, dstp)
        w, av = _tc_edge(ml, mr, att, et)
        z1 = jnp.zeros((npad, hdim), jnp.float32)
        z2 = jnp.zeros((npad, _DEN), jnp.float32)
        p1, p2 = _sc_scatter(w, av, dstp, z1, z2)
        h = _tc_finish(p1, p2, b, n, last)
    return h


# speedup vs baseline: 4.3512x; 4.3512x over previous
import functools

import jax
import jax.numpy as jnp
from jax import lax
from jax.experimental import pallas as pl
from jax.experimental.pallas import tpu as pltpu
from jax.experimental.pallas import tpu_sc as plsc

_NC = 2
_NS = 16
_NW = _NC * _NS
_B = 128
_W = 128
_DEN = 16
_GP = 3
_NPAD_SUB = 640


def _tc_mm2_pair(x, wl, wr):
    n, d = x.shape
    h = wl.shape[1]
    nb = 1000

    def body(x_ref, wl_ref, wr_ref, xl_ref, xr_ref):
        xv = x_ref[...]
        xl_ref[...] = jnp.dot(xv, wl_ref[...], preferred_element_type=jnp.float32)
        xr_ref[...] = jnp.dot(xv, wr_ref[...], preferred_element_type=jnp.float32)

    return pl.pallas_call(
        body,
        grid=(n // nb,),
        in_specs=[
            pl.BlockSpec((nb, d), lambda i: (i, 0)),
            pl.BlockSpec((d, h), lambda i: (0, 0)),
            pl.BlockSpec((d, h), lambda i: (0, 0)),
        ],
        out_specs=[
            pl.BlockSpec((nb, h), lambda i: (i, 0)),
            pl.BlockSpec((nb, h), lambda i: (i, 0)),
        ],
        out_shape=[jax.ShapeDtypeStruct((n, h), jnp.float32)] * 2,
    )(x, wl, wr)


def _mm2c_body(x_ref, wl_ref, wr_ref, o_ref, *, h):
    x = x_ref[...]
    o_ref[:, :h] = jnp.dot(x, wl_ref[...], preferred_element_type=jnp.float32)
    o_ref[:, h:2 * h] = jnp.dot(x, wr_ref[...],
                                preferred_element_type=jnp.float32)
    if 2 * h < _W:
        o_ref[:, 2 * h:] = jnp.zeros_like(o_ref[:, 2 * h:])


def _tc_mm2_combined(x, wl, wr):
    n, d = x.shape
    h = wl.shape[1]
    nb = 1000
    body = functools.partial(_mm2c_body, h=h)
    return pl.pallas_call(
        body,
        grid=(n // nb,),
        in_specs=[
            pl.BlockSpec((nb, d), lambda i: (i, 0)),
            pl.BlockSpec((d, h), lambda i: (0, 0)),
            pl.BlockSpec((d, h), lambda i: (0, 0)),
        ],
        out_specs=pl.BlockSpec((nb, _W), lambda i: (i, 0)),
        out_shape=jax.ShapeDtypeStruct((n, _W), jnp.float32),
    )(x, wl, wr)


def _edge1_body(gs_ref, gd_ref, att_ref, w_ref, aw_ref, *, e_total, eb):
    i = pl.program_id(0)
    gs = gs_ref[...]
    m = gs + gd_ref[...]
    m = jnp.where(m > 0, m, 0.2 * m)
    e = jnp.sum(m * att_ref[...], axis=1, keepdims=True)
    rows = i * eb + lax.broadcasted_iota(jnp.int32, (eb, 1), 0)
    a = jnp.where(rows < e_total, jnp.exp(e), 0.0)
    w_ref[...] = a * gs
    aw_ref[...] = jnp.broadcast_to(a, (eb, _W))


def _tc_edge1(gs, gd, att, e_total):
    ep = gs.shape[0]
    eb = 4096
    body = functools.partial(_edge1_body, e_total=e_total, eb=eb)
    return pl.pallas_call(
        body,
        grid=(ep // eb,),
        in_specs=[
            pl.BlockSpec((eb, _W), lambda i: (i, 0)),
            pl.BlockSpec((eb, _W), lambda i: (i, 0)),
            pl.BlockSpec((1, _W), lambda i: (0, 0)),
        ],
        out_specs=[
            pl.BlockSpec((eb, _W), lambda i: (i, 0)),
            pl.BlockSpec((eb, _W), lambda i: (i, 0)),
        ],
        out_shape=[jax.ShapeDtypeStruct((ep, _W), jnp.float32)] * 2,
    )(gs, gd, att.reshape(1, _W))


def _edge23_body(gs_ref, gd_ref, att_ref, w_ref, *, e_total, eb, h):
    i = pl.program_id(0)
    xl = gs_ref[:, :h]
    m = xl + gd_ref[:, h:2 * h]
    m = jnp.where(m > 0, m, 0.2 * m)
    e = jnp.sum(m * att_ref[:, :h], axis=1, keepdims=True)
    rows = i * eb + lax.broadcasted_iota(jnp.int32, (eb, 1), 0)
    a = jnp.where(rows < e_total, jnp.exp(e), 0.0)
    w_ref[:, :h] = a * xl
    w_ref[:, h:h + _DEN] = jnp.broadcast_to(a, (eb, _DEN))
    w_ref[:, h + _DEN:] = jnp.zeros_like(w_ref[:, h + _DEN:])


def _tc_edge23(gs, gd, att, e_total, h):
    ep = gs.shape[0]
    eb = 4096
    body = functools.partial(_edge23_body, e_total=e_total, eb=eb, h=h)
    att_p = jnp.zeros((1, _W), jnp.float32).at[0, :h].set(att)
    return pl.pallas_call(
        body,
        grid=(ep // eb,),
        in_specs=[
            pl.BlockSpec((eb, _W), lambda i: (i, 0)),
            pl.BlockSpec((eb, _W), lambda i: (i, 0)),
            pl.BlockSpec((1, _W), lambda i: (0, 0)),
        ],
        out_specs=pl.BlockSpec((eb, _W), lambda i: (i, 0)),
        out_shape=jax.ShapeDtypeStruct((ep, _W), jnp.float32),
    )(gs, gd, att_p)


def _finish1_body(o1_ref, o2_ref, b_ref, o_ref):
    out = o1_ref[...] / o2_ref[:, 0:1] + b_ref[...]
    o_ref[...] = jnp.maximum(out, 0.0)


def _tc_finish1(o1, o2, b, n):
    nb = 1000
    blk = pl.BlockSpec((nb, _W), lambda i: (i, 0))
    return pl.pallas_call(
        _finish1_body,
        grid=(n // nb,),
        in_specs=[blk, blk, pl.BlockSpec((1, _W), lambda i: (0, 0))],
        out_specs=blk,
        out_shape=jax.ShapeDtypeStruct((n, _W), jnp.float32),
    )(o1, o2, b.reshape(1, _W))


def _finish23_body(p_ref, b_ref, o_ref, *, h, last):
    acc = p_ref[0] + p_ref[1]
    out = acc[:, :h] / acc[:, h:h + 1] + b_ref[...]
    if last:
        mx = jnp.max(out, axis=1, keepdims=True)
        s = out - mx
        out = s - jnp.log(jnp.sum(jnp.exp(s), axis=1, keepdims=True))
    else:
        out = jnp.maximum(out, 0.0)
    o_ref[...] = out


def _tc_finish23(p, b, n, h, last):
    nb = 1000
    body = functools.partial(_finish23_body, h=h, last=last)
    return pl.pallas_call(
        body,
        grid=(n // nb,),
        in_specs=[
            pl.BlockSpec((2, nb, _W), lambda i: (0, i, 0)),
            pl.BlockSpec((1, h), lambda i: (0, 0)),
        ],
        out_specs=pl.BlockSpec((nb, h), lambda i: (i, 0)),
        out_shape=jax.ShapeDtypeStruct((n, h), jnp.float32),
    )(p, b.reshape(1, h))


def _sc_gather(tab_a, tab_b, src3, dst3):
    chunks = src3.shape[1]
    ep = _NW * chunks * _B
    mesh = plsc.VectorSubcoreMesh(core_axis_name="c", subcore_axis_name="s")
    row_t = pltpu.VMEM((_B, _W), jnp.float32)

    @functools.partial(
        pl.kernel,
        out_type=[jax.ShapeDtypeStruct((ep, _W), jnp.float32)] * 2,
        mesh=mesh,
        scratch_types=(
            [pltpu.VMEM((chunks, _B), jnp.int32)] * 2
            + [row_t] * (2 * _GP)
            + [pltpu.SemaphoreType.DMA] * (4 * _GP)
        ),
    )
    def k(ta_hbm, tb_hbm, si_hbm, di_hbm, gs_hbm, gd_hbm, si_v, di_v, *bufs):
        bl = bufs[0:_GP]
        br = bufs[_GP:2 * _GP]
        gl = bufs[2 * _GP:3 * _GP]
        gr = bufs[3 * _GP:4 * _GP]
        wl = bufs[4 * _GP:5 * _GP]
        wr = bufs[5 * _GP:6 * _GP]
        wid = lax.axis_index("s") * _NC + lax.axis_index("c")
        row0 = wid * chunks
        pltpu.sync_copy(si_hbm.at[wid], si_v)
        pltpu.sync_copy(di_hbm.at[wid], di_v)

        @pl.loop(0, chunks, step=_GP)
        def _(i0):
            hg = []
            for j in range(_GP):
                hg.append(pltpu.async_copy(ta_hbm.at[si_v.at[i0 + j]],
                                           bl[j], gl[j]))
                hg.append(pltpu.async_copy(tb_hbm.at[di_v.at[i0 + j]],
                                           br[j], gr[j]))
            hw = []
            for j in range(_GP):
                base = (row0 + i0 + j) * _B
                hg[2 * j].wait()
                hw.append(pltpu.async_copy(bl[j], gs_hbm.at[pl.ds(base, _B)],
                                           wl[j]))
                hg[2 * j + 1].wait()
                hw.append(pltpu.async_copy(br[j], gd_hbm.at[pl.ds(base, _B)],
                                           wr[j]))
            for hh in hw:
                hh.wait()

    return k(tab_a, tab_b, src3, dst3)


def _sc_scatter_dual(w, aw, dstd, z):
    ep = w.shape[0]
    npad = z.shape[0]
    halves = dstd.shape[1]
    chunks = dstd.shape[2]
    mesh = plsc.VectorSubcoreMesh(core_axis_name="c", subcore_axis_name="s")

    @functools.partial(
        pl.kernel,
        out_type=[jax.ShapeDtypeStruct((npad, _W), jnp.float32)] * 2,
        mesh=mesh,
        scratch_types=[
            pltpu.VMEM((chunks, _B), jnp.int32),
            pltpu.VMEM_SHARED((npad, _W), jnp.float32),
            pltpu.VMEM((_B, _W), jnp.float32),
            pltpu.VMEM((_B, _W), jnp.float32),
            pltpu.SemaphoreType.DMA, pltpu.SemaphoreType.DMA,
            pltpu.SemaphoreType.DMA, pltpu.SemaphoreType.DMA,
        ],
    )
    def k(w_hbm, aw_hbm, di_hbm, z_hbm, o1_hbm, o2_hbm, di_v, acc,
          wv0, wv1, ls0, ls1, ss0, ss1):
        cid = lax.axis_index("c")
        sid = lax.axis_index("s")
        r0 = sid * _NPAD_SUB
        pltpu.sync_copy(z_hbm.at[pl.ds(r0, _NPAD_SUB)],
                        acc.at[pl.ds(r0, _NPAD_SUB)])
        plsc.subcore_barrier()
        wv, lsem, ssem = (wv0, wv1), (ls0, ls1), (ss0, ss1)

        def scat(src_hbm):
            for half in range(halves):
                pltpu.sync_copy(di_hbm.at[sid, half], di_v)
                row0 = (sid * halves + half) * chunks

                @pl.loop(0, chunks, step=2)
                def _(i0):
                    hl = []
                    for j in range(2):
                        base = (row0 + i0 + j) * _B
                        hl.append(pltpu.async_copy(
                            src_hbm.at[pl.ds(base, _B)], wv[j], lsem[j]))
                    hs = []
                    for j in range(2):
                        hl[j].wait()
                        hs.append(pltpu.async_copy(
                            wv[j], acc.at[di_v.at[i0 + j]], ssem[j], add=True))
                    for hh in hs:
                        hh.wait()

        @pl.when(cid == 0)
        def _():
            scat(w_hbm)

        @pl.when(cid == 1)
        def _():
            scat(aw_hbm)

        plsc.subcore_barrier()

        @pl.when(cid == 0)
        def _():
            pltpu.sync_copy(acc.at[pl.ds(r0, _NPAD_SUB)],
                            o1_hbm.at[pl.ds(r0, _NPAD_SUB)])

        @pl.when(cid == 1)
        def _():
            pltpu.sync_copy(acc.at[pl.ds(r0, _NPAD_SUB)],
                            o2_hbm.at[pl.ds(r0, _NPAD_SUB)])

    return k(w, aw, dstd, z)


def _sc_scatter_half(w, dst3, z):
    ep = w.shape[0]
    npad = z.shape[0]
    chunks = dst3.shape[1]
    mesh = plsc.VectorSubcoreMesh(core_axis_name="c", subcore_axis_name="s")

    @functools.partial(
        pl.kernel,
        out_type=jax.ShapeDtypeStruct((_NC, npad, _W), jnp.float32),
        mesh=mesh,
        scratch_types=[
            pltpu.VMEM((chunks, _B), jnp.int32),
            pltpu.VMEM_SHARED((npad, _W), jnp.float32),
            pltpu.VMEM((_B, _W), jnp.float32),
            pltpu.VMEM((_B, _W), jnp.float32),
            pltpu.SemaphoreType.DMA, pltpu.SemaphoreType.DMA,
            pltpu.SemaphoreType.DMA, pltpu.SemaphoreType.DMA,
        ],
    )
    def k(w_hbm, di_hbm, z_hbm, o_hbm, di_v, acc, wv0, wv1, ls0, ls1, ss0, ss1):
        cid = lax.axis_index("c")
        sid = lax.axis_index("s")
        r0 = sid * _NPAD_SUB
        pltpu.sync_copy(z_hbm.at[pl.ds(r0, _NPAD_SUB)],
                        acc.at[pl.ds(r0, _NPAD_SUB)])
        wid = sid * _NC + cid
        row0 = wid * chunks
        pltpu.sync_copy(di_hbm.at[wid], di_v)
        plsc.subcore_barrier()
        wv, lsem, ssem = (wv0, wv1), (ls0, ls1), (ss0, ss1)

        @pl.loop(0, chunks, step=2)
        def _(i0):
            hl = []
            for j in range(2):
                base = (row0 + i0 + j) * _B
                hl.append(pltpu.async_copy(
                    w_hbm.at[pl.ds(base, _B)], wv[j], lsem[j]))
            hs = []
            for j in range(2):
                hl[j].wait()
                hs.append(pltpu.async_copy(
                    wv[j], acc.at[di_v.at[i0 + j]], ssem[j], add=True))
            for hh in hs:
                hh.wait()

        plsc.subcore_barrier()
        pltpu.sync_copy(acc.at[pl.ds(r0, _NPAD_SUB)],
                        o_hbm.at[cid, pl.ds(r0, _NPAD_SUB)])

    return k(w, dst3, z)


def kernel(x, edge_index, W1l, W1r, a1, b1, W2l, W2r, a2, b2, W3l, W3r, a3, b3):
    n = x.shape[0]
    e = edge_index.shape[1]
    npad = _NS * _NPAD_SUB
    loops = jnp.arange(n, dtype=edge_index.dtype)
    src = jnp.concatenate([edge_index[0], loops])
    dst = jnp.concatenate([edge_index[1], loops])
    et = e + n
    grain = _NW * _B * _GP * 2
    ep = ((et + grain - 1) // grain) * grain
    pad = ep - et
    srcp = jnp.concatenate([src, jnp.zeros((pad,), src.dtype)])
    dstp = jnp.concatenate([dst, jnp.zeros((pad,), dst.dtype)])
    src3 = srcp.reshape(_NW, -1, _B)
    dst3 = dstp.reshape(_NW, -1, _B)
    dstd = dstp.reshape(_NS, 2, -1, _B)
    z = jnp.zeros((npad, _W), jnp.float32)

    xl, xr = _tc_mm2_pair(x, W1l, W1r)
    gs, gd = _sc_gather(xl, xr, src3, dst3)
    w, aw = _tc_edge1(gs, gd, a1, et)
    o1, o2 = _sc_scatter_dual(w, aw, dstd, z)
    h = _tc_finish1(o1, o2, b1, n)

    for wl, wr, att, b, last in ((W2l, W2r, a2, b2, False),
                                 (W3l, W3r, a3, b3, True)):
        hdim = wl.shape[1]
        t = _tc_mm2_combined(h, wl, wr)
        gs, gd = _sc_gather(t, t, src3, dst3)
        w = _tc_edge23(gs, gd, att, et, hdim)
        p = _sc_scatter_half(w, dst3, z)
        h = _tc_finish23(p, b, n, hdim, last)
    return h


# speedup vs baseline: 4.8691x; 1.1190x over previous
import dataclasses
import functools

import jax
import jax.numpy as jnp
from jax import lax
from jax.experimental import pallas as pl
from jax.experimental.pallas import tpu as pltpu
from jax.experimental.pallas import tpu_sc as plsc

_NC = 2
_NS = 16
_NW = _NC * _NS
_B = 128
_W = 128
_DEN = 16
_GP = 2
_NPAD_SUB = 640


def _sc_compiler_params():
    cp = pltpu.CompilerParams()
    if "needs_layout_passes" in pltpu.CompilerParams.__dataclass_fields__:
        cp = dataclasses.replace(cp, needs_layout_passes=False)
    return cp


def _tc_mm2_pair(x, wl, wr):
    n, d = x.shape
    h = wl.shape[1]
    nb = 1000

    def body(x_ref, wl_ref, wr_ref, xl_ref, xr_ref):
        xv = x_ref[...]
        xl_ref[...] = jnp.dot(xv, wl_ref[...], preferred_element_type=jnp.float32)
        xr_ref[...] = jnp.dot(xv, wr_ref[...], preferred_element_type=jnp.float32)

    return pl.pallas_call(
        body,
        grid=(n // nb,),
        in_specs=[
            pl.BlockSpec((nb, d), lambda i: (i, 0)),
            pl.BlockSpec((d, h), lambda i: (0, 0)),
            pl.BlockSpec((d, h), lambda i: (0, 0)),
        ],
        out_specs=[
            pl.BlockSpec((nb, h), lambda i: (i, 0)),
            pl.BlockSpec((nb, h), lambda i: (i, 0)),
        ],
        out_shape=[jax.ShapeDtypeStruct((n, h), jnp.float32)] * 2,
    )(x, wl, wr)


def _mm2c_body(x_ref, wl_ref, wr_ref, o_ref, *, h):
    x = x_ref[...]
    o_ref[:, :h] = jnp.dot(x, wl_ref[...], preferred_element_type=jnp.float32)
    o_ref[:, h:2 * h] = jnp.dot(x, wr_ref[...],
                                preferred_element_type=jnp.float32)
    if 2 * h < _W:
        o_ref[:, 2 * h:] = jnp.zeros_like(o_ref[:, 2 * h:])


def _tc_mm2_combined(x, wl, wr):
    n, d = x.shape
    h = wl.shape[1]
    nb = 1000
    body = functools.partial(_mm2c_body, h=h)
    return pl.pallas_call(
        body,
        grid=(n // nb,),
        in_specs=[
            pl.BlockSpec((nb, d), lambda i: (i, 0)),
            pl.BlockSpec((d, h), lambda i: (0, 0)),
            pl.BlockSpec((d, h), lambda i: (0, 0)),
        ],
        out_specs=pl.BlockSpec((nb, _W), lambda i: (i, 0)),
        out_shape=jax.ShapeDtypeStruct((n, _W), jnp.float32),
    )(x, wl, wr)


def _finish1_body(o1_ref, o2_ref, b_ref, o_ref):
    out = o1_ref[...] / o2_ref[:, 0:1] + b_ref[...]
    o_ref[...] = jnp.maximum(out, 0.0)


def _tc_finish1(o1, o2, b, n):
    nb = 1000
    blk = pl.BlockSpec((nb, _W), lambda i: (i, 0))
    return pl.pallas_call(
        _finish1_body,
        grid=(n // nb,),
        in_specs=[blk, blk, pl.BlockSpec((1, _W), lambda i: (0, 0))],
        out_specs=blk,
        out_shape=jax.ShapeDtypeStruct((n, _W), jnp.float32),
    )(o1, o2, b.reshape(1, _W))


def _finish23_body(p_ref, b_ref, o_ref, *, h, last):
    acc = p_ref[0] + p_ref[1]
    out = acc[:, :h] / acc[:, h:h + 1] + b_ref[...]
    if last:
        mx = jnp.max(out, axis=1, keepdims=True)
        s = out - mx
        out = s - jnp.log(jnp.sum(jnp.exp(s), axis=1, keepdims=True))
    else:
        out = jnp.maximum(out, 0.0)
    o_ref[...] = out


def _tc_finish23(p, b, n, h, last):
    nb = 1000
    body = functools.partial(_finish23_body, h=h, last=last)
    return pl.pallas_call(
        body,
        grid=(n // nb,),
        in_specs=[
            pl.BlockSpec((2, nb, _W), lambda i: (0, i, 0)),
            pl.BlockSpec((1, h), lambda i: (0, 0)),
        ],
        out_specs=pl.BlockSpec((nb, h), lambda i: (i, 0)),
        out_shape=jax.ShapeDtypeStruct((n, h), jnp.float32),
    )(p, b.reshape(1, h))


def _sc_fused23(tab, attv, src3, dst3, h):
    halves = src3.shape[1]
    chunks = src3.shape[2]
    bs = src3.shape[3]
    ep = _NW * halves * chunks * bs
    nq = h // 16
    mesh = plsc.VectorSubcoreMesh(core_axis_name="c", subcore_axis_name="s")
    row_t = pltpu.VMEM((bs, _W), jnp.float32)

    @functools.partial(
        pl.kernel,
        out_type=jax.ShapeDtypeStruct((ep, _W), jnp.float32),
        mesh=mesh,
        compiler_params=_sc_compiler_params(),
        scratch_types=(
            [pltpu.VMEM((chunks, bs), jnp.int32)] * 2
            + [row_t] * 4
            + [pltpu.VMEM((8, 16), jnp.float32)]
            + [pltpu.SemaphoreType.DMA] * 6
        ),
    )
    def k(t_hbm, att_hbm, si_hbm, di_hbm, w_hbm, si_v, di_v,
          bl0, bl1, br0, br1, att_v, gl0, gl1, gr0, gr1, ws0, ws1):
        wid = lax.axis_index("s") * _NC + lax.axis_index("c")
        pltpu.sync_copy(att_hbm, att_v)
        atts = [att_v[q, :] for q in range(nq)]
        bl, br = (bl0, bl1), (br0, br1)
        gl, gr, ws = (gl0, gl1), (gr0, gr1), (ws0, ws1)

        for half in range(halves):
            pltpu.sync_copy(si_hbm.at[wid, half], si_v)
            pltpu.sync_copy(di_hbm.at[wid, half], di_v)
            row0 = (wid * halves + half) * chunks

            @pl.loop(0, chunks, step=2)
            def _(i0):
                hg = []
                for j in range(2):
                    hg.append(pltpu.async_copy(t_hbm.at[si_v.at[i0 + j]],
                                               bl[j], gl[j]))
                    hg.append(pltpu.async_copy(t_hbm.at[di_v.at[i0 + j]],
                                               br[j], gr[j]))
                hw = []
                for j in range(2):
                    hg[2 * j].wait()
                    hg[2 * j + 1].wait()
                    blj, brj = bl[j], br[j]

                    @pl.loop(0, bs)
                    def _(r):
                        us = []
                        acc = jnp.zeros((16,), jnp.float32)
                        for q in range(nq):
                            u = blj[r, pl.ds(16 * q, 16)]
                            v = brj[r, pl.ds(h + 16 * q, 16)]
                            us.append(u)
                            mq = u + v
                            mq = jnp.where(mq > 0, mq, 0.2 * mq)
                            acc = acc + mq * atts[q]
                        e = jnp.sum(acc)
                        av = jnp.exp(lax.broadcast_in_dim(e, (16,), ()))
                        for q in range(nq):
                            blj[r, pl.ds(16 * q, 16)] = us[q] * av
                        blj[r, pl.ds(h, 16)] = av

                    base = (row0 + i0 + j) * bs
                    hw.append(pltpu.async_copy(blj,
                                               w_hbm.at[pl.ds(base, bs)],
                                               ws[j]))
                for hh in hw:
                    hh.wait()

    return k(tab, attv, src3, dst3)


def _sc_fused1(tab_l, tab_r, attv, src3, dst3):
    halves = src3.shape[1]
    chunks = src3.shape[2]
    bs = src3.shape[3]
    ep = _NW * halves * chunks * bs
    mesh = plsc.VectorSubcoreMesh(core_axis_name="c", subcore_axis_name="s")
    row_t = pltpu.VMEM((bs, _W), jnp.float32)

    @functools.partial(
        pl.kernel,
        out_type=[jax.ShapeDtypeStruct((ep, _W), jnp.float32),
                  jax.ShapeDtypeStruct((ep, _DEN), jnp.float32)],
        mesh=mesh,
        compiler_params=_sc_compiler_params(),
        scratch_types=(
            [pltpu.VMEM((chunks, bs), jnp.int32)] * 2
            + [row_t] * 4
            + [pltpu.VMEM((bs, _DEN), jnp.float32)] * 2
            + [pltpu.VMEM((8, 16), jnp.float32)]
            + [pltpu.SemaphoreType.DMA] * 8
        ),
    )
    def k(tl_hbm, tr_hbm, att_hbm, si_hbm, di_hbm, w_hbm, a_hbm, si_v, di_v,
          bl0, bl1, br0, br1, av0, av1, att_v,
          gl0, gl1, gr0, gr1, ws0, ws1, as0, as1):
        wid = lax.axis_index("s") * _NC + lax.axis_index("c")
        pltpu.sync_copy(att_hbm, att_v)
        atts = [att_v[q, :] for q in range(8)]
        bl, br = (bl0, bl1), (br0, br1)
        avb = (av0, av1)
        gl, gr, ws, asem = (gl0, gl1), (gr0, gr1), (ws0, ws1), (as0, as1)

        for half in range(halves):
            pltpu.sync_copy(si_hbm.at[wid, half], si_v)
            pltpu.sync_copy(di_hbm.at[wid, half], di_v)
            row0 = (wid * halves + half) * chunks

            @pl.loop(0, chunks, step=2)
            def _(i0):
                hg = []
                for j in range(2):
                    hg.append(pltpu.async_copy(tl_hbm.at[si_v.at[i0 + j]],
                                               bl[j], gl[j]))
                    hg.append(pltpu.async_copy(tr_hbm.at[di_v.at[i0 + j]],
                                               br[j], gr[j]))
                hw = []
                for j in range(2):
                    hg[2 * j].wait()
                    hg[2 * j + 1].wait()
                    blj, brj, avj = bl[j], br[j], avb[j]

                    @pl.loop(0, bs)
                    def _(r):
                        us = []
                        acc = jnp.zeros((16,), jnp.float32)
                        for q in range(8):
                            u = blj[r, pl.ds(16 * q, 16)]
                            v = brj[r, pl.ds(16 * q, 16)]
                            us.append(u)
                            mq = u + v
                            mq = jnp.where(mq > 0, mq, 0.2 * mq)
                            acc = acc + mq * atts[q]
                        e = jnp.sum(acc)
                        av = jnp.exp(lax.broadcast_in_dim(e, (16,), ()))
                        for q in range(8):
                            blj[r, pl.ds(16 * q, 16)] = us[q] * av
                        avj[r, :] = av

                    base = (row0 + i0 + j) * bs
                    hw.append(pltpu.async_copy(blj,
                                               w_hbm.at[pl.ds(base, bs)],
                                               ws[j]))
                    hw.append(pltpu.async_copy(avj,
                                               a_hbm.at[pl.ds(base, bs)],
                                               asem[j]))
                for hh in hw:
                    hh.wait()

    return k(tab_l, tab_r, attv, src3, dst3)


def _sc_scatter_dual(w, a16r, dstd, z):
    ep = w.shape[0]
    npad = z.shape[0]
    halves = dstd.shape[1]
    chunks = dstd.shape[2]
    bs = dstd.shape[3]
    mesh = plsc.VectorSubcoreMesh(core_axis_name="c", subcore_axis_name="s")

    @functools.partial(
        pl.kernel,
        out_type=[jax.ShapeDtypeStruct((npad, _W), jnp.float32)] * 2,
        mesh=mesh,
        compiler_params=_sc_compiler_params(),
        scratch_types=[
            pltpu.VMEM((chunks, bs), jnp.int32),
            pltpu.VMEM_SHARED((npad, _W), jnp.float32),
            pltpu.VMEM((bs, _W), jnp.float32),
            pltpu.VMEM((bs, _W), jnp.float32),
            pltpu.VMEM((bs // 8, _W), jnp.float32),
            pltpu.VMEM((bs // 8, _W), jnp.float32),
            pltpu.SemaphoreType.DMA, pltpu.SemaphoreType.DMA,
            pltpu.SemaphoreType.DMA, pltpu.SemaphoreType.DMA,
        ],
    )
    def k(w_hbm, a_hbm, di_hbm, z_hbm, o1_hbm, o2_hbm, di_v, acc,
          wv0, wv1, av0, av1, ls0, ls1, ss0, ss1):
        cid = lax.axis_index("c")
        sid = lax.axis_index("s")
        r0 = sid * _NPAD_SUB
        pltpu.sync_copy(z_hbm.at[pl.ds(r0, _NPAD_SUB)],
                        acc.at[pl.ds(r0, _NPAD_SUB)])
        plsc.subcore_barrier()
        wv, avb = (wv0, wv1), (av0, av1)
        lsem, ssem = (ls0, ls1), (ss0, ss1)

        @pl.when(cid == 0)
        def _():
            for half in range(halves):
                pltpu.sync_copy(di_hbm.at[sid, half], di_v)
                row0 = (sid * halves + half) * chunks

                @pl.loop(0, chunks, step=2)
                def _(i0):
                    hl = []
                    for j in range(2):
                        base = (row0 + i0 + j) * bs
                        hl.append(pltpu.async_copy(
                            w_hbm.at[pl.ds(base, bs)], wv[j], lsem[j]))
                    hs = []
                    for j in range(2):
                        hl[j].wait()
                        hs.append(pltpu.async_copy(
                            wv[j], acc.at[di_v.at[i0 + j]], ssem[j], add=True))
                    for hh in hs:
                        hh.wait()

        @pl.when(cid == 1)
        def _():
            for half in range(halves):
                pltpu.sync_copy(di_hbm.at[sid, half], di_v)
                row0 = (sid * halves + half) * chunks

                @pl.loop(0, chunks, step=2)
                def _(i0):
                    ha = []
                    for j in range(2):
                        base8 = (row0 + i0 + j) * (bs // 8)
                        ha.append(pltpu.async_copy(
                            a_hbm.at[pl.ds(base8, bs // 8)], avb[j], lsem[j]))
                    hs = []
                    for j in range(2):
                        ha[j].wait()
                        avj, wvj = avb[j], wv[j]

                        @pl.loop(0, bs // 8)
                        def _(ra):
                            for kk in range(8):
                                wvj[ra * 8 + kk, pl.ds(0, _DEN)] = (
                                    avj[ra, pl.ds(16 * kk, _DEN)])

                        hs.append(pltpu.async_copy(
                            wvj, acc.at[di_v.at[i0 + j]], ssem[j], add=True))
                    for hh in hs:
                        hh.wait()

        plsc.subcore_barrier()

        @pl.when(cid == 0)
        def _():
            pltpu.sync_copy(acc.at[pl.ds(r0, _NPAD_SUB)],
                            o1_hbm.at[pl.ds(r0, _NPAD_SUB)])

        @pl.when(cid == 1)
        def _():
            pltpu.sync_copy(acc.at[pl.ds(r0, _NPAD_SUB)],
                            o2_hbm.at[pl.ds(r0, _NPAD_SUB)])

    return k(w, a16r, dstd, z)


def _sc_scatter_half(w, dst3, z):
    npad = z.shape[0]
    chunks = dst3.shape[1]
    bs = dst3.shape[2]
    mesh = plsc.VectorSubcoreMesh(core_axis_name="c", subcore_axis_name="s")

    @functools.partial(
        pl.kernel,
        out_type=jax.ShapeDtypeStruct((_NC, npad, _W), jnp.float32),
        mesh=mesh,
        scratch_types=[
            pltpu.VMEM((chunks, bs), jnp.int32),
            pltpu.VMEM_SHARED((npad, _W), jnp.float32),
            pltpu.VMEM((bs, _W), jnp.float32),
            pltpu.VMEM((bs, _W), jnp.float32),
            pltpu.SemaphoreType.DMA, pltpu.SemaphoreType.DMA,
            pltpu.SemaphoreType.DMA, pltpu.SemaphoreType.DMA,
        ],
    )
    def k(w_hbm, di_hbm, z_hbm, o_hbm, di_v, acc, wv0, wv1, ls0, ls1, ss0, ss1):
        cid = lax.axis_index("c")
        sid = lax.axis_index("s")
        r0 = sid * _NPAD_SUB
        pltpu.sync_copy(z_hbm.at[pl.ds(r0, _NPAD_SUB)],
                        acc.at[pl.ds(r0, _NPAD_SUB)])
        wid = sid * _NC + cid
        row0 = wid * chunks
        pltpu.sync_copy(di_hbm.at[wid], di_v)
        plsc.subcore_barrier()
        wv, lsem, ssem = (wv0, wv1), (ls0, ls1), (ss0, ss1)

        @pl.loop(0, chunks, step=2)
        def _(i0):
            hl = []
            for j in range(2):
                base = (row0 + i0 + j) * bs
                hl.append(pltpu.async_copy(
                    w_hbm.at[pl.ds(base, bs)], wv[j], lsem[j]))
            hs = []
            for j in range(2):
                hl[j].wait()
                hs.append(pltpu.async_copy(
                    wv[j], acc.at[di_v.at[i0 + j]], ssem[j], add=True))
            for hh in hs:
                hh.wait()

        plsc.subcore_barrier()
        pltpu.sync_copy(acc.at[pl.ds(r0, _NPAD_SUB)],
                        o_hbm.at[cid, pl.ds(r0, _NPAD_SUB)])

    return k(w, dst3, z)


def kernel(x, edge_index, W1l, W1r, a1, b1, W2l, W2r, a2, b2, W3l, W3r, a3, b3):
    n = x.shape[0]
    e = edge_index.shape[1]
    npad = _NS * _NPAD_SUB
    loops = jnp.arange(n, dtype=edge_index.dtype)
    src = jnp.concatenate([edge_index[0], loops])
    dst = jnp.concatenate([edge_index[1], loops])
    et = e + n
    grain = _NW * _B * 4
    ep = ((et + grain - 1) // grain) * grain
    pad = ep - et
    srcp = jnp.concatenate([src, jnp.zeros((pad,), src.dtype)])
    dstp = jnp.concatenate([dst, jnp.full((pad,), npad - 1, dst.dtype)])
    src4 = srcp.reshape(_NW, 4, -1, 64)
    dst4 = dstp.reshape(_NW, 4, -1, 64)
    dst3 = dstp.reshape(_NW, -1, 64)
    dstd = dstp.reshape(_NS, 2, -1, 64)
    z = jnp.zeros((npad, _W), jnp.float32)

    xl, xr = _tc_mm2_pair(x, W1l, W1r)
    w, a16 = _sc_fused1(xl, xr, a1.reshape(8, 16), src4, dst4)
    o1, o2 = _sc_scatter_dual(w, a16.reshape(-1, _W), dstd, z)
    h = _tc_finish1(o1, o2, b1, n)

    for wl, wr, att, b, last in ((W2l, W2r, a2, b2, False),
                                 (W3l, W3r, a3, b3, True)):
        hdim = wl.shape[1]
        t = _tc_mm2_combined(h, wl, wr)
        att_p = jnp.zeros((_W,), jnp.float32).at[:hdim].set(att).reshape(8, 16)
        w = _sc_fused23(t, att_p, src4, dst4, hdim)
        p = _sc_scatter_half(w, dst3, z)
        h = _tc_finish23(p, b, n, hdim, last)
    return h


# speedup vs baseline: 5.2794x; 1.0843x over previous
import dataclasses
import functools

import jax
import jax.numpy as jnp
from jax import lax
from jax.experimental import pallas as pl
from jax.experimental.pallas import tpu as pltpu
from jax.experimental.pallas import tpu_sc as plsc

_NC = 2
_NS = 16
_NW = _NC * _NS
_B = 128
_W = 128
_DEN = 16
_GP = 2
_NPAD_SUB = 640


def _sc_compiler_params():
    cp = pltpu.CompilerParams()
    if "needs_layout_passes" in pltpu.CompilerParams.__dataclass_fields__:
        cp = dataclasses.replace(cp, needs_layout_passes=False)
    return cp


def _tc_mm2_pair(x, wl, wr):
    n, d = x.shape
    h = wl.shape[1]
    nb = 1000

    def body(x_ref, wl_ref, wr_ref, xl_ref, xr_ref):
        xv = x_ref[...]
        xl_ref[...] = jnp.dot(xv, wl_ref[...], preferred_element_type=jnp.float32)
        xr_ref[...] = jnp.dot(xv, wr_ref[...], preferred_element_type=jnp.float32)

    return pl.pallas_call(
        body,
        grid=(n // nb,),
        in_specs=[
            pl.BlockSpec((nb, d), lambda i: (i, 0)),
            pl.BlockSpec((d, h), lambda i: (0, 0)),
            pl.BlockSpec((d, h), lambda i: (0, 0)),
        ],
        out_specs=[
            pl.BlockSpec((nb, h), lambda i: (i, 0)),
            pl.BlockSpec((nb, h), lambda i: (i, 0)),
        ],
        out_shape=[jax.ShapeDtypeStruct((n, h), jnp.float32)] * 2,
    )(x, wl, wr)


def _mm2c_body(x_ref, wl_ref, wr_ref, o_ref, *, h):
    x = x_ref[...]
    o_ref[:, :h] = jnp.dot(x, wl_ref[...], preferred_element_type=jnp.float32)
    o_ref[:, h:2 * h] = jnp.dot(x, wr_ref[...],
                                preferred_element_type=jnp.float32)
    if 2 * h < _W:
        o_ref[:, 2 * h:] = jnp.zeros_like(o_ref[:, 2 * h:])


def _tc_mm2_combined(x, wl, wr):
    n, d = x.shape
    h = wl.shape[1]
    nb = 1000
    body = functools.partial(_mm2c_body, h=h)
    return pl.pallas_call(
        body,
        grid=(n // nb,),
        in_specs=[
            pl.BlockSpec((nb, d), lambda i: (i, 0)),
            pl.BlockSpec((d, h), lambda i: (0, 0)),
            pl.BlockSpec((d, h), lambda i: (0, 0)),
        ],
        out_specs=pl.BlockSpec((nb, _W), lambda i: (i, 0)),
        out_shape=jax.ShapeDtypeStruct((n, _W), jnp.float32),
    )(x, wl, wr)


def _finish1_body(o1_ref, o2_ref, b_ref, o_ref):
    out = o1_ref[...] / o2_ref[:, 0:1] + b_ref[...]
    o_ref[...] = jnp.maximum(out, 0.0)


def _tc_finish1(o1, o2, b, n):
    nb = 1000
    blk = pl.BlockSpec((nb, _W), lambda i: (i, 0))
    return pl.pallas_call(
        _finish1_body,
        grid=(n // nb,),
        in_specs=[blk, blk, pl.BlockSpec((1, _W), lambda i: (0, 0))],
        out_specs=blk,
        out_shape=jax.ShapeDtypeStruct((n, _W), jnp.float32),
    )(o1, o2, b.reshape(1, _W))


def _finish23_body(p_ref, b_ref, o_ref, *, h, last):
    acc = p_ref[0] + p_ref[1]
    out = acc[:, :h] / acc[:, h:h + 1] + b_ref[...]
    if last:
        mx = jnp.max(out, axis=1, keepdims=True)
        s = out - mx
        out = s - jnp.log(jnp.sum(jnp.exp(s), axis=1, keepdims=True))
    else:
        out = jnp.maximum(out, 0.0)
    o_ref[...] = out


def _tc_finish23(p, b, n, h, last):
    nb = 1000
    body = functools.partial(_finish23_body, h=h, last=last)
    return pl.pallas_call(
        body,
        grid=(n // nb,),
        in_specs=[
            pl.BlockSpec((2, nb, _W), lambda i: (0, i, 0)),
            pl.BlockSpec((1, h), lambda i: (0, 0)),
        ],
        out_specs=pl.BlockSpec((nb, h), lambda i: (i, 0)),
        out_shape=jax.ShapeDtypeStruct((n, h), jnp.float32),
    )(p, b.reshape(1, h))


def _sc_fused23(tab, attv, src3, dst3, h):
    halves = src3.shape[1]
    chunks = src3.shape[2]
    bs = src3.shape[3]
    ep = _NW * halves * chunks * bs
    nq = h // 16
    mesh = plsc.VectorSubcoreMesh(core_axis_name="c", subcore_axis_name="s")
    row_t = pltpu.VMEM((bs, _W), jnp.float32)

    @functools.partial(
        pl.kernel,
        out_type=jax.ShapeDtypeStruct((ep, _W), jnp.float32),
        mesh=mesh,
        compiler_params=_sc_compiler_params(),
        scratch_types=(
            [pltpu.VMEM((chunks, bs), jnp.int32)] * 2
            + [row_t] * 4
            + [pltpu.VMEM((8, 16), jnp.float32)]
            + [pltpu.SemaphoreType.DMA] * 6
        ),
    )
    def k(t_hbm, att_hbm, si_hbm, di_hbm, w_hbm, si_v, di_v,
          bl0, bl1, br0, br1, att_v, gl0, gl1, gr0, gr1, ws0, ws1):
        wid = lax.axis_index("s") * _NC + lax.axis_index("c")
        pltpu.sync_copy(att_hbm, att_v)
        atts = [att_v[q, :] for q in range(nq)]
        bl, br = (bl0, bl1), (br0, br1)
        gl, gr, ws = (gl0, gl1), (gr0, gr1), (ws0, ws1)

        for half in range(halves):
            pltpu.sync_copy(si_hbm.at[wid, half], si_v)
            pltpu.sync_copy(di_hbm.at[wid, half], di_v)
            row0 = (wid * halves + half) * chunks

            @pl.loop(0, chunks, step=2)
            def _(i0):
                hg = []
                for j in range(2):
                    hg.append(pltpu.async_copy(t_hbm.at[si_v.at[i0 + j]],
                                               bl[j], gl[j]))
                    hg.append(pltpu.async_copy(t_hbm.at[di_v.at[i0 + j]],
                                               br[j], gr[j]))
                hw = []
                for j in range(2):
                    hg[2 * j].wait()
                    hg[2 * j + 1].wait()
                    blj, brj = bl[j], br[j]

                    @pl.loop(0, bs)
                    def _(r):
                        us = []
                        acc = jnp.zeros((16,), jnp.float32)
                        for q in range(nq):
                            u = blj[r, pl.ds(16 * q, 16)]
                            v = brj[r, pl.ds(h + 16 * q, 16)]
                            us.append(u)
                            mq = u + v
                            mq = jnp.where(mq > 0, mq, 0.2 * mq)
                            acc = acc + mq * atts[q]
                        e = jnp.sum(acc)
                        av = jnp.exp(lax.broadcast_in_dim(e, (16,), ()))
                        for q in range(nq):
                            blj[r, pl.ds(16 * q, 16)] = us[q] * av
                        blj[r, pl.ds(h, 16)] = av

                    base = (row0 + i0 + j) * bs
                    hw.append(pltpu.async_copy(blj,
                                               w_hbm.at[pl.ds(base, bs)],
                                               ws[j]))
                for hh in hw:
                    hh.wait()

    return k(tab, attv, src3, dst3)


def _sc_fused1(tab_l, tab_r, attv, src3, dst3):
    halves = src3.shape[1]
    chunks = src3.shape[2]
    bs = src3.shape[3]
    ep = _NW * halves * chunks * bs
    mesh = plsc.VectorSubcoreMesh(core_axis_name="c", subcore_axis_name="s")
    row_t = pltpu.VMEM((bs, _W), jnp.float32)

    @functools.partial(
        pl.kernel,
        out_type=[jax.ShapeDtypeStruct((ep, _W), jnp.float32),
                  jax.ShapeDtypeStruct((ep, _DEN), jnp.float32)],
        mesh=mesh,
        compiler_params=_sc_compiler_params(),
        scratch_types=(
            [pltpu.VMEM((chunks, bs), jnp.int32)] * 2
            + [row_t] * 4
            + [pltpu.VMEM((bs, _DEN), jnp.float32)] * 2
            + [pltpu.VMEM((8, 16), jnp.float32)]
            + [pltpu.SemaphoreType.DMA] * 8
        ),
    )
    def k(tl_hbm, tr_hbm, att_hbm, si_hbm, di_hbm, w_hbm, a_hbm, si_v, di_v,
          bl0, bl1, br0, br1, av0, av1, att_v,
          gl0, gl1, gr0, gr1, ws0, ws1, as0, as1):
        wid = lax.axis_index("s") * _NC + lax.axis_index("c")
        pltpu.sync_copy(att_hbm, att_v)
        atts = [att_v[q, :] for q in range(8)]
        bl, br = (bl0, bl1), (br0, br1)
        avb = (av0, av1)
        gl, gr, ws, asem = (gl0, gl1), (gr0, gr1), (ws0, ws1), (as0, as1)

        for half in range(halves):
            pltpu.sync_copy(si_hbm.at[wid, half], si_v)
            pltpu.sync_copy(di_hbm.at[wid, half], di_v)
            row0 = (wid * halves + half) * chunks

            @pl.loop(0, chunks, step=2)
            def _(i0):
                hg = []
                for j in range(2):
                    hg.append(pltpu.async_copy(tl_hbm.at[si_v.at[i0 + j]],
                                               bl[j], gl[j]))
                    hg.append(pltpu.async_copy(tr_hbm.at[di_v.at[i0 + j]],
                                               br[j], gr[j]))
                hw = []
                for j in range(2):
                    hg[2 * j].wait()
                    hg[2 * j + 1].wait()
                    blj, brj, avj = bl[j], br[j], avb[j]

                    @pl.loop(0, bs)
                    def _(r):
                        us = []
                        acc = jnp.zeros((16,), jnp.float32)
                        for q in range(8):
                            u = blj[r, pl.ds(16 * q, 16)]
                            v = brj[r, pl.ds(16 * q, 16)]
                            us.append(u)
                            mq = u + v
                            mq = jnp.where(mq > 0, mq, 0.2 * mq)
                            acc = acc + mq * atts[q]
                        e = jnp.sum(acc)
                        av = jnp.exp(lax.broadcast_in_dim(e, (16,), ()))
                        for q in range(8):
                            blj[r, pl.ds(16 * q, 16)] = us[q] * av
                        avj[r, :] = av

                    base = (row0 + i0 + j) * bs
                    hw.append(pltpu.async_copy(blj,
                                               w_hbm.at[pl.ds(base, bs)],
                                               ws[j]))
                    hw.append(pltpu.async_copy(avj,
                                               a_hbm.at[pl.ds(base, bs)],
                                               asem[j]))
                for hh in hw:
                    hh.wait()

    return k(tab_l, tab_r, attv, src3, dst3)


def _sc_scatter_dual(w, a16r, dstd, z):
    ep = w.shape[0]
    npad = z.shape[0]
    halves = dstd.shape[1]
    chunks = dstd.shape[2]
    bs = dstd.shape[3]
    mesh = plsc.VectorSubcoreMesh(core_axis_name="c", subcore_axis_name="s")

    @functools.partial(
        pl.kernel,
        out_type=[jax.ShapeDtypeStruct((npad, _W), jnp.float32)] * 2,
        mesh=mesh,
        compiler_params=_sc_compiler_params(),
        scratch_types=[
            pltpu.VMEM((chunks, bs), jnp.int32),
            pltpu.VMEM_SHARED((npad, _W), jnp.float32),
            pltpu.VMEM((bs, _W), jnp.float32),
            pltpu.VMEM((bs, _W), jnp.float32),
            pltpu.VMEM((bs // 8, _W), jnp.float32),
            pltpu.VMEM((bs // 8, _W), jnp.float32),
            pltpu.SemaphoreType.DMA, pltpu.SemaphoreType.DMA,
            pltpu.SemaphoreType.DMA, pltpu.SemaphoreType.DMA,
        ],
    )
    def k(w_hbm, a_hbm, di_hbm, z_hbm, o1_hbm, o2_hbm, di_v, acc,
          wv0, wv1, av0, av1, ls0, ls1, ss0, ss1):
        cid = lax.axis_index("c")
        sid = lax.axis_index("s")
        r0 = sid * _NPAD_SUB
        pltpu.sync_copy(z_hbm.at[pl.ds(r0, _NPAD_SUB)],
                        acc.at[pl.ds(r0, _NPAD_SUB)])
        plsc.subcore_barrier()
        wv, avb = (wv0, wv1), (av0, av1)
        lsem, ssem = (ls0, ls1), (ss0, ss1)

        @pl.when(cid == 0)
        def _():
            for half in range(halves):
                pltpu.sync_copy(di_hbm.at[sid, half], di_v)
                row0 = (sid * halves + half) * chunks

                @pl.loop(0, chunks, step=2)
                def _(i0):
                    hl = []
                    for j in range(2):
                        base = (row0 + i0 + j) * bs
                        hl.append(pltpu.async_copy(
                            w_hbm.at[pl.ds(base, bs)], wv[j], lsem[j]))
                    hs = []
                    for j in range(2):
                        hl[j].wait()
                        hs.append(pltpu.async_copy(
                            wv[j], acc.at[di_v.at[i0 + j]], ssem[j], add=True))
                    for hh in hs:
                        hh.wait()

        @pl.when(cid == 1)
        def _():
            for half in range(halves):
                pltpu.sync_copy(di_hbm.at[sid, half], di_v)
                row0 = (sid * halves + half) * chunks

                @pl.loop(0, chunks, step=2)
                def _(i0):
                    ha = []
                    for j in range(2):
                        base8 = (row0 + i0 + j) * (bs // 8)
                        ha.append(pltpu.async_copy(
                            a_hbm.at[pl.ds(base8, bs // 8)], avb[j], lsem[j]))
                    hs = []
                    for j in range(2):
                        ha[j].wait()
                        avj, wvj = avb[j], wv[j]

                        @pl.loop(0, bs // 8)
                        def _(ra):
                            for kk in range(8):
                                wvj[ra * 8 + kk, pl.ds(0, _DEN)] = (
                                    avj[ra, pl.ds(16 * kk, _DEN)])

                        hs.append(pltpu.async_copy(
                            wvj, acc.at[di_v.at[i0 + j]], ssem[j], add=True))
                    for hh in hs:
                        hh.wait()

        plsc.subcore_barrier()

        @pl.when(cid == 0)
        def _():
            pltpu.sync_copy(acc.at[pl.ds(r0, _NPAD_SUB)],
                            o1_hbm.at[pl.ds(r0, _NPAD_SUB)])

        @pl.when(cid == 1)
        def _():
            pltpu.sync_copy(acc.at[pl.ds(r0, _NPAD_SUB)],
                            o2_hbm.at[pl.ds(r0, _NPAD_SUB)])

    return k(w, a16r, dstd, z)


def _sc_scatter_half(w, dst3, z):
    npad = z.shape[0]
    chunks = dst3.shape[1]
    bs = dst3.shape[2]
    mesh = plsc.VectorSubcoreMesh(core_axis_name="c", subcore_axis_name="s")

    @functools.partial(
        pl.kernel,
        out_type=jax.ShapeDtypeStruct((_NC, npad, _W), jnp.float32),
        mesh=mesh,
        scratch_types=[
            pltpu.VMEM((chunks, bs), jnp.int32),
            pltpu.VMEM_SHARED((npad, _W), jnp.float32),
            pltpu.VMEM((bs, _W), jnp.float32),
            pltpu.VMEM((bs, _W), jnp.float32),
            pltpu.SemaphoreType.DMA, pltpu.SemaphoreType.DMA,
            pltpu.SemaphoreType.DMA, pltpu.SemaphoreType.DMA,
        ],
    )
    def k(w_hbm, di_hbm, z_hbm, o_hbm, di_v, acc, wv0, wv1, ls0, ls1, ss0, ss1):
        cid = lax.axis_index("c")
        sid = lax.axis_index("s")
        r0 = sid * _NPAD_SUB
        pltpu.sync_copy(z_hbm.at[pl.ds(r0, _NPAD_SUB)],
                        acc.at[pl.ds(r0, _NPAD_SUB)])
        wid = sid * _NC + cid
        row0 = wid * chunks
        pltpu.sync_copy(di_hbm.at[wid], di_v)
        plsc.subcore_barrier()
        wv, lsem, ssem = (wv0, wv1), (ls0, ls1), (ss0, ss1)

        @pl.loop(0, chunks, step=2)
        def _(i0):
            hl = []
            for j in range(2):
                base = (row0 + i0 + j) * bs
                hl.append(pltpu.async_copy(
                    w_hbm.at[pl.ds(base, bs)], wv[j], lsem[j]))
            hs = []
            for j in range(2):
                hl[j].wait()
                hs.append(pltpu.async_copy(
                    wv[j], acc.at[di_v.at[i0 + j]], ssem[j], add=True))
            for hh in hs:
                hh.wait()

        plsc.subcore_barrier()
        pltpu.sync_copy(acc.at[pl.ds(r0, _NPAD_SUB)],
                        o_hbm.at[cid, pl.ds(r0, _NPAD_SUB)])

    return k(w, dst3, z)


def kernel(x, edge_index, W1l, W1r, a1, b1, W2l, W2r, a2, b2, W3l, W3r, a3, b3):
    n = x.shape[0]
    e = edge_index.shape[1]
    npad = _NS * _NPAD_SUB
    loops = jnp.arange(n, dtype=edge_index.dtype)
    src = jnp.concatenate([edge_index[0], loops])
    dst = jnp.concatenate([edge_index[1], loops])
    et = e + n
    grain = _NW * _B * 4
    ep = ((et + grain - 1) // grain) * grain
    pad = ep - et
    srcp = jnp.concatenate([src, jnp.zeros((pad,), src.dtype)])
    dstp = jnp.concatenate([dst, jnp.full((pad,), npad - 1, dst.dtype)])
    src4 = srcp.reshape(_NW, 2, -1, _B)
    dst4 = dstp.reshape(_NW, 2, -1, _B)
    dst3 = dstp.reshape(_NW, -1, _B)
    dstd = dstp.reshape(_NS, 2, -1, _B)
    z = jnp.zeros((npad, _W), jnp.float32)

    xl, xr = _tc_mm2_pair(x, W1l, W1r)
    w, a16 = _sc_fused1(xl, xr, a1.reshape(8, 16), src4, dst4)
    o1, o2 = _sc_scatter_dual(w, a16.reshape(-1, _W), dstd, z)
    h = _tc_finish1(o1, o2, b1, n)

    for wl, wr, att, b, last in ((W2l, W2r, a2, b2, False),
                                 (W3l, W3r, a3, b3, True)):
        hdim = wl.shape[1]
        t = _tc_mm2_combined(h, wl, wr)
        att_p = jnp.zeros((_W,), jnp.float32).at[:hdim].set(att).reshape(8, 16)
        w = _sc_fused23(t, att_p, src4, dst4, hdim)
        p = _sc_scatter_half(w, dst3, z)
        h = _tc_finish23(p, b, n, hdim, last)
    return h


# speedup vs baseline: 7.9251x; 1.5011x over previous
import dataclasses
import functools

import jax
import jax.numpy as jnp
from jax import lax
from jax.experimental import pallas as pl
from jax.experimental.pallas import tpu as pltpu
from jax.experimental.pallas import tpu_sc as plsc

_NC = 2
_NS = 16
_NW = _NC * _NS
_B = 128
_W = 128
_DEN = 16
_GP = 2
_NPAD_SUB = 640


def _sc_compiler_params():
    cp = pltpu.CompilerParams()
    if "needs_layout_passes" in pltpu.CompilerParams.__dataclass_fields__:
        cp = dataclasses.replace(cp, needs_layout_passes=False)
    return cp


def _tc_mm2_pair(x, wl, wr):
    n, d = x.shape
    h = wl.shape[1]
    nb = 1000

    def body(x_ref, wl_ref, wr_ref, xl_ref, xr_ref):
        xv = x_ref[...]
        xl_ref[...] = jnp.dot(xv, wl_ref[...], preferred_element_type=jnp.float32)
        xr_ref[...] = jnp.dot(xv, wr_ref[...], preferred_element_type=jnp.float32)

    return pl.pallas_call(
        body,
        grid=(n // nb,),
        in_specs=[
            pl.BlockSpec((nb, d), lambda i: (i, 0)),
            pl.BlockSpec((d, h), lambda i: (0, 0)),
            pl.BlockSpec((d, h), lambda i: (0, 0)),
        ],
        out_specs=[
            pl.BlockSpec((nb, h), lambda i: (i, 0)),
            pl.BlockSpec((nb, h), lambda i: (i, 0)),
        ],
        out_shape=[jax.ShapeDtypeStruct((n, h), jnp.float32)] * 2,
    )(x, wl, wr)


def _mm2c_body(x_ref, wl_ref, wr_ref, o_ref, *, h):
    x = x_ref[...]
    o_ref[:, :h] = jnp.dot(x, wl_ref[...], preferred_element_type=jnp.float32)
    o_ref[:, h:2 * h] = jnp.dot(x, wr_ref[...],
                                preferred_element_type=jnp.float32)
    if 2 * h < _W:
        o_ref[:, 2 * h:] = jnp.zeros_like(o_ref[:, 2 * h:])


def _tc_mm2_combined(x, wl, wr):
    n, d = x.shape
    h = wl.shape[1]
    nb = 1000
    body = functools.partial(_mm2c_body, h=h)
    return pl.pallas_call(
        body,
        grid=(n // nb,),
        in_specs=[
            pl.BlockSpec((nb, d), lambda i: (i, 0)),
            pl.BlockSpec((d, h), lambda i: (0, 0)),
            pl.BlockSpec((d, h), lambda i: (0, 0)),
        ],
        out_specs=pl.BlockSpec((nb, _W), lambda i: (i, 0)),
        out_shape=jax.ShapeDtypeStruct((n, _W), jnp.float32),
    )(x, wl, wr)


def _edge1_body(gs_ref, gd_ref, att_ref, w_ref, aw_ref, *, e_total, eb):
    i = pl.program_id(0)
    gs = gs_ref[...]
    m = gs + gd_ref[...].astype(jnp.float32)
    m = jnp.where(m > 0, m, 0.2 * m)
    e = jnp.sum(m * att_ref[...], axis=1, keepdims=True)
    rows = i * eb + lax.broadcasted_iota(jnp.int32, (eb, 1), 0)
    a = jnp.where(rows < e_total, jnp.exp(e), 0.0)
    w_ref[...] = a * gs
    aw_ref[...] = jnp.broadcast_to(a, (eb, _W))


def _tc_edge1(gs, gd, att, e_total):
    ep = gs.shape[0]
    eb = 4096
    body = functools.partial(_edge1_body, e_total=e_total, eb=eb)
    return pl.pallas_call(
        body,
        grid=(ep // eb,),
        in_specs=[
            pl.BlockSpec((eb, _W), lambda i: (i, 0)),
            pl.BlockSpec((eb, _W), lambda i: (i, 0)),
            pl.BlockSpec((1, _W), lambda i: (0, 0)),
        ],
        out_specs=[
            pl.BlockSpec((eb, _W), lambda i: (i, 0)),
            pl.BlockSpec((eb, _W), lambda i: (i, 0)),
        ],
        out_shape=[jax.ShapeDtypeStruct((ep, _W), jnp.float32)] * 2,
    )(gs, gd, att.reshape(1, _W))


def _edge23_body(gs_ref, gd_ref, att_ref, w_ref, *, e_total, eb, h):
    i = pl.program_id(0)
    xl = gs_ref[:, :h]
    m = xl + gd_ref[:, h:2 * h].astype(jnp.float32)
    m = jnp.where(m > 0, m, 0.2 * m)
    e = jnp.sum(m * att_ref[:, :h], axis=1, keepdims=True)
    rows = i * eb + lax.broadcasted_iota(jnp.int32, (eb, 1), 0)
    a = jnp.where(rows < e_total, jnp.exp(e), 0.0)
    w_ref[:, :h] = a * xl
    w_ref[:, h:h + _DEN] = jnp.broadcast_to(a, (eb, _DEN))
    w_ref[:, h + _DEN:] = jnp.zeros_like(w_ref[:, h + _DEN:])


def _tc_edge23(gs, gd, att, e_total, h):
    ep = gs.shape[0]
    eb = 4096
    body = functools.partial(_edge23_body, e_total=e_total, eb=eb, h=h)
    att_p = jnp.zeros((1, _W), jnp.float32).at[0, :h].set(att)
    return pl.pallas_call(
        body,
        grid=(ep // eb,),
        in_specs=[
            pl.BlockSpec((eb, _W), lambda i: (i, 0)),
            pl.BlockSpec((eb, _W), lambda i: (i, 0)),
            pl.BlockSpec((1, _W), lambda i: (0, 0)),
        ],
        out_specs=pl.BlockSpec((eb, _W), lambda i: (i, 0)),
        out_shape=jax.ShapeDtypeStruct((ep, _W), jnp.float32),
    )(gs, gd, att_p)


def _finish1_body(o1_ref, o2_ref, b_ref, o_ref):
    out = o1_ref[...] / o2_ref[:, 0:1] + b_ref[...]
    o_ref[...] = jnp.maximum(out, 0.0)


def _tc_finish1(o1, o2, b, n):
    nb = 1000
    blk = pl.BlockSpec((nb, _W), lambda i: (i, 0))
    return pl.pallas_call(
        _finish1_body,
        grid=(n // nb,),
        in_specs=[blk, blk, pl.BlockSpec((1, _W), lambda i: (0, 0))],
        out_specs=blk,
        out_shape=jax.ShapeDtypeStruct((n, _W), jnp.float32),
    )(o1, o2, b.reshape(1, _W))


def _finish23_body(p_ref, b_ref, o_ref, *, h, last):
    acc = p_ref[0] + p_ref[1]
    out = acc[:, :h] / acc[:, h:h + 1] + b_ref[...]
    if last:
        mx = jnp.max(out, axis=1, keepdims=True)
        s = out - mx
        out = s - jnp.log(jnp.sum(jnp.exp(s), axis=1, keepdims=True))
    else:
        out = jnp.maximum(out, 0.0)
    o_ref[...] = out


def _tc_finish23(p, b, n, h, last):
    nb = 1000
    body = functools.partial(_finish23_body, h=h, last=last)
    return pl.pallas_call(
        body,
        grid=(n // nb,),
        in_specs=[
            pl.BlockSpec((2, nb, _W), lambda i: (0, i, 0)),
            pl.BlockSpec((1, h), lambda i: (0, 0)),
        ],
        out_specs=pl.BlockSpec((nb, h), lambda i: (i, 0)),
        out_shape=jax.ShapeDtypeStruct((n, h), jnp.float32),
    )(p, b.reshape(1, h))


def _sc_gather(tab_a, tab_b, src3, dst3):
    chunks = src3.shape[1]
    ep = _NW * chunks * _B
    mesh = plsc.VectorSubcoreMesh(core_axis_name="c", subcore_axis_name="s")
    row_t = pltpu.VMEM((_B, _W), jnp.float32)

    @functools.partial(
        pl.kernel,
        out_type=[jax.ShapeDtypeStruct((ep, _W), jnp.float32)] * 2,
        mesh=mesh,
        scratch_types=(
            [pltpu.VMEM((chunks, _B), jnp.int32)] * 2
            + [row_t] * (2 * _GP)
            + [pltpu.SemaphoreType.DMA] * (4 * _GP)
        ),
    )
    def k(ta_hbm, tb_hbm, si_hbm, di_hbm, gs_hbm, gd_hbm, si_v, di_v, *bufs):
        bl = bufs[0:_GP]
        br = bufs[_GP:2 * _GP]
        gl = bufs[2 * _GP:3 * _GP]
        gr = bufs[3 * _GP:4 * _GP]
        wl = bufs[4 * _GP:5 * _GP]
        wr = bufs[5 * _GP:6 * _GP]
        wid = lax.axis_index("s") * _NC + lax.axis_index("c")
        row0 = wid * chunks
        pltpu.sync_copy(si_hbm.at[wid], si_v)
        pltpu.sync_copy(di_hbm.at[wid], di_v)

        @pl.loop(0, chunks, step=_GP)
        def _(i0):
            hg = []
            for j in range(_GP):
                hg.append(pltpu.async_copy(ta_hbm.at[si_v.at[i0 + j]],
                                           bl[j], gl[j]))
                hg.append(pltpu.async_copy(tb_hbm.at[di_v.at[i0 + j]],
                                           br[j], gr[j]))
            hw = []
            for j in range(_GP):
                base = (row0 + i0 + j) * _B
                hg[2 * j].wait()
                hw.append(pltpu.async_copy(bl[j], gs_hbm.at[pl.ds(base, _B)],
                                           wl[j]))
                hg[2 * j + 1].wait()
                hw.append(pltpu.async_copy(br[j], gd_hbm.at[pl.ds(base, _B)],
                                           wr[j]))
            for hh in hw:
                hh.wait()

    return k(tab_a, tab_b, src3, dst3)


def _sc_fused23(tab, attv, src3, dst3, h):
    chunks = src3.shape[1]
    ep = _NW * chunks * _B
    nq = h // 16
    mesh = plsc.VectorSubcoreMesh(core_axis_name="c", subcore_axis_name="s")
    row_t = pltpu.VMEM((_B, _W), jnp.float32)

    @functools.partial(
        pl.kernel,
        out_type=jax.ShapeDtypeStruct((ep, _W), jnp.float32),
        mesh=mesh,
        compiler_params=_sc_compiler_params(),
        scratch_types=(
            [pltpu.VMEM((chunks, _B), jnp.int32)] * 2
            + [row_t] * 6
            + [pltpu.VMEM((8, 16), jnp.float32)]
            + [pltpu.SemaphoreType.DMA] * 6
        ),
    )
    def k(t_hbm, att_hbm, si_hbm, di_hbm, w_hbm, si_v, di_v,
          bl0, bl1, br0, br1, wv0, wv1, att_v, gl0, gl1, gr0, gr1, ws0, ws1):
        wid = lax.axis_index("s") * _NC + lax.axis_index("c")
        row0 = wid * chunks
        pltpu.sync_copy(si_hbm.at[wid], si_v)
        pltpu.sync_copy(di_hbm.at[wid], di_v)
        pltpu.sync_copy(att_hbm, att_v)
        atts = [att_v[q, :] for q in range(nq)]
        bl, br, wv = (bl0, bl1), (br0, br1), (wv0, wv1)
        gl, gr, ws = (gl0, gl1), (gr0, gr1), (ws0, ws1)

        @pl.loop(0, chunks, step=2)
        def _(i0):
            hg = []
            for j in range(2):
                hg.append(pltpu.async_copy(t_hbm.at[si_v.at[i0 + j]],
                                           bl[j], gl[j]))
                hg.append(pltpu.async_copy(t_hbm.at[di_v.at[i0 + j]],
                                           br[j], gr[j]))
            hw = []
            for j in range(2):
                hg[2 * j].wait()
                hg[2 * j + 1].wait()
                blj, brj, wvj = bl[j], br[j], wv[j]

                @pl.loop(0, _B)
                def _(r):
                    us = []
                    acc = jnp.zeros((16,), jnp.float32)
                    for q in range(nq):
                        u = blj[r, pl.ds(16 * q, 16)]
                        v = brj[r, pl.ds(h + 16 * q, 16)]
                        us.append(u)
                        mq = u + v
                        mq = jnp.where(mq > 0, mq, 0.2 * mq)
                        acc = acc + mq * atts[q]
                    e = jnp.sum(acc)
                    av = jnp.exp(lax.broadcast_in_dim(e, (16,), ()))
                    for q in range(nq):
                        wvj[r, pl.ds(16 * q, 16)] = us[q] * av
                    wvj[r, pl.ds(h, 16)] = av

                base = (row0 + i0 + j) * _B
                hw.append(pltpu.async_copy(wvj, w_hbm.at[pl.ds(base, _B)],
                                           ws[j]))
            for hh in hw:
                hh.wait()

    return k(tab, attv, src3, dst3)


def _sc_scatter_dual(w, aw, dstd, z):
    ep = w.shape[0]
    npad = z.shape[0]
    halves = dstd.shape[1]
    chunks = dstd.shape[2]
    mesh = plsc.VectorSubcoreMesh(core_axis_name="c", subcore_axis_name="s")

    @functools.partial(
        pl.kernel,
        out_type=[jax.ShapeDtypeStruct((npad, _W), jnp.float32)] * 2,
        mesh=mesh,
        scratch_types=[
            pltpu.VMEM((chunks, _B), jnp.int32),
            pltpu.VMEM_SHARED((npad, _W), jnp.float32),
            pltpu.VMEM((_B, _W), jnp.float32),
            pltpu.VMEM((_B, _W), jnp.float32),
            pltpu.SemaphoreType.DMA, pltpu.SemaphoreType.DMA,
            pltpu.SemaphoreType.DMA, pltpu.SemaphoreType.DMA,
        ],
    )
    def k(w_hbm, aw_hbm, di_hbm, z_hbm, o1_hbm, o2_hbm, di_v, acc,
          wv0, wv1, ls0, ls1, ss0, ss1):
        cid = lax.axis_index("c")
        sid = lax.axis_index("s")
        r0 = sid * _NPAD_SUB
        pltpu.sync_copy(z_hbm.at[pl.ds(r0, _NPAD_SUB)],
                        acc.at[pl.ds(r0, _NPAD_SUB)])
        plsc.subcore_barrier()
        wv, lsem, ssem = (wv0, wv1), (ls0, ls1), (ss0, ss1)

        def scat(src_hbm):
            for half in range(halves):
                pltpu.sync_copy(di_hbm.at[sid, half], di_v)
                row0 = (sid * halves + half) * chunks

                @pl.loop(0, chunks, step=2)
                def _(i0):
                    hl = []
                    for j in range(2):
                        base = (row0 + i0 + j) * _B
                        hl.append(pltpu.async_copy(
                            src_hbm.at[pl.ds(base, _B)], wv[j], lsem[j]))
                    hs = []
                    for j in range(2):
                        hl[j].wait()
                        hs.append(pltpu.async_copy(
                            wv[j], acc.at[di_v.at[i0 + j]], ssem[j], add=True))
                    for hh in hs:
                        hh.wait()

        @pl.when(cid == 0)
        def _():
            scat(w_hbm)

        @pl.when(cid == 1)
        def _():
            scat(aw_hbm)

        plsc.subcore_barrier()

        @pl.when(cid == 0)
        def _():
            pltpu.sync_copy(acc.at[pl.ds(r0, _NPAD_SUB)],
                            o1_hbm.at[pl.ds(r0, _NPAD_SUB)])

        @pl.when(cid == 1)
        def _():
            pltpu.sync_copy(acc.at[pl.ds(r0, _NPAD_SUB)],
                            o2_hbm.at[pl.ds(r0, _NPAD_SUB)])

    return k(w, aw, dstd, z)


def _sc_scatter_half(w, dst3, z):
    ep = w.shape[0]
    npad = z.shape[0]
    chunks = dst3.shape[1]
    mesh = plsc.VectorSubcoreMesh(core_axis_name="c", subcore_axis_name="s")

    @functools.partial(
        pl.kernel,
        out_type=jax.ShapeDtypeStruct((_NC, npad, _W), jnp.float32),
        mesh=mesh,
        scratch_types=[
            pltpu.VMEM((chunks, _B), jnp.int32),
            pltpu.VMEM_SHARED((npad, _W), jnp.float32),
            pltpu.VMEM((_B, _W), jnp.float32),
            pltpu.VMEM((_B, _W), jnp.float32),
            pltpu.SemaphoreType.DMA, pltpu.SemaphoreType.DMA,
            pltpu.SemaphoreType.DMA, pltpu.SemaphoreType.DMA,
        ],
    )
    def k(w_hbm, di_hbm, z_hbm, o_hbm, di_v, acc, wv0, wv1, ls0, ls1, ss0, ss1):
        cid = lax.axis_index("c")
        sid = lax.axis_index("s")
        r0 = sid * _NPAD_SUB
        pltpu.sync_copy(z_hbm.at[pl.ds(r0, _NPAD_SUB)],
                        acc.at[pl.ds(r0, _NPAD_SUB)])
        wid = sid * _NC + cid
        row0 = wid * chunks
        pltpu.sync_copy(di_hbm.at[wid], di_v)
        plsc.subcore_barrier()
        wv, lsem, ssem = (wv0, wv1), (ls0, ls1), (ss0, ss1)

        @pl.loop(0, chunks, step=2)
        def _(i0):
            hl = []
            for j in range(2):
                base = (row0 + i0 + j) * _B
                hl.append(pltpu.async_copy(
                    w_hbm.at[pl.ds(base, _B)], wv[j], lsem[j]))
            hs = []
            for j in range(2):
                hl[j].wait()
                hs.append(pltpu.async_copy(
                    wv[j], acc.at[di_v.at[i0 + j]], ssem[j], add=True))
            for hh in hs:
                hh.wait()

        plsc.subcore_barrier()
        pltpu.sync_copy(acc.at[pl.ds(r0, _NPAD_SUB)],
                        o_hbm.at[cid, pl.ds(r0, _NPAD_SUB)])

    return k(w, dst3, z)


def kernel(x, edge_index, W1l, W1r, a1, b1, W2l, W2r, a2, b2, W3l, W3r, a3, b3):
    n = x.shape[0]
    e = edge_index.shape[1]
    npad = _NS * _NPAD_SUB
    loops = jnp.arange(n, dtype=edge_index.dtype)
    src = jnp.concatenate([edge_index[0], loops])
    dst = jnp.concatenate([edge_index[1], loops])
    et = e + n
    grain = _NW * _B * _GP
    ep = ((et + grain - 1) // grain) * grain
    pad = ep - et
    srcp = jnp.concatenate([src, jnp.zeros((pad,), src.dtype)])
    dstp = jnp.concatenate([dst, jnp.full((pad,), npad - 1, dst.dtype)])
    src3 = srcp.reshape(_NW, -1, _B)
    dst3 = dstp.reshape(_NW, -1, _B)
    dstd = dstp.reshape(_NS, 2, -1, _B)
    z = jnp.zeros((npad, _W), jnp.float32)

    xl, xr = _tc_mm2_pair(x, W1l, W1r)
    gs, gd = _sc_gather(xl, xr, src3, dst3)
    w, aw = _tc_edge1(gs, gd, a1, et)
    o1, o2 = _sc_scatter_dual(w, aw, dstd, z)
    h = _tc_finish1(o1, o2, b1, n)

    for wl, wr, att, b, last in ((W2l, W2r, a2, b2, False),
                                 (W3l, W3r, a3, b3, True)):
        hdim = wl.shape[1]
        t = _tc_mm2_combined(h, wl, wr)
        att_p = jnp.zeros((_W,), jnp.float32).at[:hdim].set(att).reshape(8, 16)
        w = _sc_fused23(t, att_p, src3, dst3, hdim)
        p = _sc_scatter_half(w, dst3, z)
        h = _tc_finish23(p, b, n, hdim, last)
    return h


# speedup vs baseline: 7.9375x; 1.0016x over previous
import dataclasses
import functools

import jax
import jax.numpy as jnp
from jax import lax
from jax.experimental import pallas as pl
from jax.experimental.pallas import tpu as pltpu
from jax.experimental.pallas import tpu_sc as plsc

_NC = 2
_NS = 16
_NW = _NC * _NS
_B = 128
_W = 128
_DEN = 16
_GP = 2
_NPAD_SUB = 640


def _sc_compiler_params():
    cp = pltpu.CompilerParams()
    if "needs_layout_passes" in pltpu.CompilerParams.__dataclass_fields__:
        cp = dataclasses.replace(cp, needs_layout_passes=False)
    return cp


def _tc_mm2_pair(x, wl, wr):
    n, d = x.shape
    h = wl.shape[1]
    nb = 1000

    def body(x_ref, wl_ref, wr_ref, xl_ref, xr_ref):
        xv = x_ref[...]
        xl_ref[...] = jnp.dot(xv, wl_ref[...], preferred_element_type=jnp.float32)
        xr_ref[...] = jnp.dot(xv, wr_ref[...], preferred_element_type=jnp.float32)

    return pl.pallas_call(
        body,
        grid=(n // nb,),
        in_specs=[
            pl.BlockSpec((nb, d), lambda i: (i, 0)),
            pl.BlockSpec((d, h), lambda i: (0, 0)),
            pl.BlockSpec((d, h), lambda i: (0, 0)),
        ],
        out_specs=[
            pl.BlockSpec((nb, h), lambda i: (i, 0)),
            pl.BlockSpec((nb, h), lambda i: (i, 0)),
        ],
        out_shape=[jax.ShapeDtypeStruct((n, h), jnp.float32)] * 2,
    )(x, wl, wr)


def _mm2c_body(x_ref, wl_ref, wr_ref, o_ref, *, h):
    x = x_ref[...]
    o_ref[:, :h] = jnp.dot(x, wl_ref[...], preferred_element_type=jnp.float32)
    o_ref[:, h:2 * h] = jnp.dot(x, wr_ref[...],
                                preferred_element_type=jnp.float32)
    if 2 * h < _W:
        o_ref[:, 2 * h:] = jnp.zeros_like(o_ref[:, 2 * h:])


def _tc_mm2_combined(x, wl, wr):
    n, d = x.shape
    h = wl.shape[1]
    nb = 1000
    body = functools.partial(_mm2c_body, h=h)
    return pl.pallas_call(
        body,
        grid=(n // nb,),
        in_specs=[
            pl.BlockSpec((nb, d), lambda i: (i, 0)),
            pl.BlockSpec((d, h), lambda i: (0, 0)),
            pl.BlockSpec((d, h), lambda i: (0, 0)),
        ],
        out_specs=pl.BlockSpec((nb, _W), lambda i: (i, 0)),
        out_shape=jax.ShapeDtypeStruct((n, _W), jnp.float32),
    )(x, wl, wr)


def _edge1_body(gs_ref, gd_ref, att_ref, w_ref, aw_ref, *, e_total, eb):
    i = pl.program_id(0)
    gs = gs_ref[...]
    m = gs + gd_ref[...].astype(jnp.float32)
    m = jnp.where(m > 0, m, 0.2 * m)
    e = jnp.sum(m * att_ref[...], axis=1, keepdims=True)
    rows = i * eb + lax.broadcasted_iota(jnp.int32, (eb, 1), 0)
    a = jnp.where(rows < e_total, jnp.exp(e), 0.0)
    w_ref[...] = a * gs
    aw_ref[...] = jnp.broadcast_to(a, (eb, _W))


def _tc_edge1(gs, gd, att, e_total):
    ep = gs.shape[0]
    eb = 4096
    body = functools.partial(_edge1_body, e_total=e_total, eb=eb)
    return pl.pallas_call(
        body,
        grid=(ep // eb,),
        in_specs=[
            pl.BlockSpec((eb, _W), lambda i: (i, 0)),
            pl.BlockSpec((eb, _W), lambda i: (i, 0)),
            pl.BlockSpec((1, _W), lambda i: (0, 0)),
        ],
        out_specs=[
            pl.BlockSpec((eb, _W), lambda i: (i, 0)),
            pl.BlockSpec((eb, _W), lambda i: (i, 0)),
        ],
        out_shape=[jax.ShapeDtypeStruct((ep, _W), jnp.float32)] * 2,
    )(gs, gd, att.reshape(1, _W))


def _finish1_body(o1_ref, o2_ref, b_ref, o_ref):
    out = o1_ref[...] / o2_ref[:, 0:1] + b_ref[...]
    o_ref[...] = jnp.maximum(out, 0.0)


def _tc_finish1(o1, o2, b, n):
    nb = 1000
    blk = pl.BlockSpec((nb, _W), lambda i: (i, 0))
    return pl.pallas_call(
        _finish1_body,
        grid=(n // nb,),
        in_specs=[blk, blk, pl.BlockSpec((1, _W), lambda i: (0, 0))],
        out_specs=blk,
        out_shape=jax.ShapeDtypeStruct((n, _W), jnp.float32),
    )(o1, o2, b.reshape(1, _W))


def _finish23_body(p_ref, b_ref, o_ref, *, h, last):
    acc = p_ref[0] + p_ref[1]
    out = acc[:, :h] / acc[:, h:h + 1] + b_ref[...]
    if last:
        mx = jnp.max(out, axis=1, keepdims=True)
        s = out - mx
        out = s - jnp.log(jnp.sum(jnp.exp(s), axis=1, keepdims=True))
    else:
        out = jnp.maximum(out, 0.0)
    o_ref[...] = out


def _tc_finish23(p, b, n, h, last):
    nb = 1000
    body = functools.partial(_finish23_body, h=h, last=last)
    return pl.pallas_call(
        body,
        grid=(n // nb,),
        in_specs=[
            pl.BlockSpec((2, nb, _W), lambda i: (0, i, 0)),
            pl.BlockSpec((1, h), lambda i: (0, 0)),
        ],
        out_specs=pl.BlockSpec((nb, h), lambda i: (i, 0)),
        out_shape=jax.ShapeDtypeStruct((n, h), jnp.float32),
    )(p, b.reshape(1, h))


def _sc_gather(tab_a, tab_b, src3, dst3):
    chunks = src3.shape[1]
    ep = _NW * chunks * _B
    mesh = plsc.VectorSubcoreMesh(core_axis_name="c", subcore_axis_name="s")
    row_t = pltpu.VMEM((_B, _W), jnp.float32)

    @functools.partial(
        pl.kernel,
        out_type=[jax.ShapeDtypeStruct((ep, _W), jnp.float32)] * 2,
        mesh=mesh,
        scratch_types=(
            [pltpu.VMEM((chunks, _B), jnp.int32)] * 2
            + [row_t] * (2 * _GP)
            + [pltpu.SemaphoreType.DMA] * (4 * _GP)
        ),
    )
    def k(ta_hbm, tb_hbm, si_hbm, di_hbm, gs_hbm, gd_hbm, si_v, di_v, *bufs):
        bl = bufs[0:_GP]
        br = bufs[_GP:2 * _GP]
        gl = bufs[2 * _GP:3 * _GP]
        gr = bufs[3 * _GP:4 * _GP]
        wl = bufs[4 * _GP:5 * _GP]
        wr = bufs[5 * _GP:6 * _GP]
        wid = lax.axis_index("s") * _NC + lax.axis_index("c")
        row0 = wid * chunks
        pltpu.sync_copy(si_hbm.at[wid], si_v)
        pltpu.sync_copy(di_hbm.at[wid], di_v)

        @pl.loop(0, chunks, step=_GP)
        def _(i0):
            hg = []
            for j in range(_GP):
                hg.append(pltpu.async_copy(ta_hbm.at[si_v.at[i0 + j]],
                                           bl[j], gl[j]))
                hg.append(pltpu.async_copy(tb_hbm.at[di_v.at[i0 + j]],
                                           br[j], gr[j]))
            hw = []
            for j in range(_GP):
                base = (row0 + i0 + j) * _B
                hg[2 * j].wait()
                hw.append(pltpu.async_copy(bl[j], gs_hbm.at[pl.ds(base, _B)],
                                           wl[j]))
                hg[2 * j + 1].wait()
                hw.append(pltpu.async_copy(br[j], gd_hbm.at[pl.ds(base, _B)],
                                           wr[j]))
            for hh in hw:
                hh.wait()

    return k(tab_a, tab_b, src3, dst3)


def _sc_fused23(tab, attv, src3, dst3, h):
    chunks = src3.shape[1]
    ep = _NW * chunks * _B
    nq = h // 16
    mesh = plsc.VectorSubcoreMesh(core_axis_name="c", subcore_axis_name="s")
    row_t = pltpu.VMEM((_B, _W), jnp.float32)

    @functools.partial(
        pl.kernel,
        out_type=jax.ShapeDtypeStruct((ep, _W), jnp.float32),
        mesh=mesh,
        compiler_params=_sc_compiler_params(),
        scratch_types=(
            [pltpu.VMEM((chunks, _B), jnp.int32)] * 2
            + [row_t] * 6
            + [pltpu.VMEM((8, 16), jnp.float32)]
            + [pltpu.SemaphoreType.DMA] * 6
        ),
    )
    def k(t_hbm, att_hbm, si_hbm, di_hbm, w_hbm, si_v, di_v,
          bl0, bl1, br0, br1, wv0, wv1, att_v, gl0, gl1, gr0, gr1, ws0, ws1):
        wid = lax.axis_index("s") * _NC + lax.axis_index("c")
        row0 = wid * chunks
        pltpu.sync_copy(si_hbm.at[wid], si_v)
        pltpu.sync_copy(di_hbm.at[wid], di_v)
        pltpu.sync_copy(att_hbm, att_v)
        atts = [att_v[q, :] for q in range(nq)]
        bl, br, wv = (bl0, bl1), (br0, br1), (wv0, wv1)
        gl, gr, ws = (gl0, gl1), (gr0, gr1), (ws0, ws1)

        @pl.loop(0, chunks, step=2)
        def _(i0):
            hg = []
            for j in range(2):
                hg.append(pltpu.async_copy(t_hbm.at[si_v.at[i0 + j]],
                                           bl[j], gl[j]))
                hg.append(pltpu.async_copy(t_hbm.at[di_v.at[i0 + j]],
                                           br[j], gr[j]))
            hw = []
            for j in range(2):
                hg[2 * j].wait()
                hg[2 * j + 1].wait()
                blj, brj, wvj = bl[j], br[j], wv[j]

                @pl.loop(0, _B)
                def _(r):
                    us = []
                    acc = jnp.zeros((16,), jnp.float32)
                    for q in range(nq):
                        u = blj[r, pl.ds(16 * q, 16)]
                        v = brj[r, pl.ds(h + 16 * q, 16)]
                        us.append(u)
                        mq = u + v
                        mq = jnp.where(mq > 0, mq, 0.2 * mq)
                        acc = acc + mq * atts[q]
                    e = jnp.sum(acc)
                    av = jnp.exp(lax.broadcast_in_dim(e, (16,), ()))
                    for q in range(nq):
                        wvj[r, pl.ds(16 * q, 16)] = us[q] * av
                    wvj[r, pl.ds(h, 16)] = av

                base = (row0 + i0 + j) * _B
                hw.append(pltpu.async_copy(wvj, w_hbm.at[pl.ds(base, _B)],
                                           ws[j]))
            for hh in hw:
                hh.wait()

    return k(tab, attv, src3, dst3)


def _sc_scatter_dual(w, aw, dstd, z):
    ep = w.shape[0]
    npad = z.shape[0]
    halves = dstd.shape[1]
    chunks = dstd.shape[2]
    mesh = plsc.VectorSubcoreMesh(core_axis_name="c", subcore_axis_name="s")

    @functools.partial(
        pl.kernel,
        out_type=[jax.ShapeDtypeStruct((npad, _W), jnp.float32)] * 2,
        mesh=mesh,
        scratch_types=[
            pltpu.VMEM((chunks, _B), jnp.int32),
            pltpu.VMEM_SHARED((npad, _W), jnp.float32),
            pltpu.VMEM((_B, _W), jnp.float32),
            pltpu.VMEM((_B, _W), jnp.float32),
            pltpu.SemaphoreType.DMA, pltpu.SemaphoreType.DMA,
            pltpu.SemaphoreType.DMA, pltpu.SemaphoreType.DMA,
        ],
    )
    def k(w_hbm, aw_hbm, di_hbm, z_hbm, o1_hbm, o2_hbm, di_v, acc,
          wv0, wv1, ls0, ls1, ss0, ss1):
        cid = lax.axis_index("c")
        sid = lax.axis_index("s")
        r0 = sid * _NPAD_SUB
        pltpu.sync_copy(z_hbm.at[pl.ds(r0, _NPAD_SUB)],
                        acc.at[pl.ds(r0, _NPAD_SUB)])
        plsc.subcore_barrier()
        wv, lsem, ssem = (wv0, wv1), (ls0, ls1), (ss0, ss1)

        def scat(src_hbm):
            for half in range(halves):
                pltpu.sync_copy(di_hbm.at[sid, half], di_v)
                row0 = (sid * halves + half) * chunks

                @pl.loop(0, chunks, step=2)
                def _(i0):
                    hl = []
                    for j in range(2):
                        base = (row0 + i0 + j) * _B
                        hl.append(pltpu.async_copy(
                            src_hbm.at[pl.ds(base, _B)], wv[j], lsem[j]))
                    hs = []
                    for j in range(2):
                        hl[j].wait()
                        hs.append(pltpu.async_copy(
                            wv[j], acc.at[di_v.at[i0 + j]], ssem[j], add=True))
                    for hh in hs:
                        hh.wait()

        @pl.when(cid == 0)
        def _():
            scat(w_hbm)

        @pl.when(cid == 1)
        def _():
            scat(aw_hbm)

        plsc.subcore_barrier()

        @pl.when(cid == 0)
        def _():
            pltpu.sync_copy(acc.at[pl.ds(r0, _NPAD_SUB)],
                            o1_hbm.at[pl.ds(r0, _NPAD_SUB)])

        @pl.when(cid == 1)
        def _():
            pltpu.sync_copy(acc.at[pl.ds(r0, _NPAD_SUB)],
                            o2_hbm.at[pl.ds(r0, _NPAD_SUB)])

    return k(w, aw, dstd, z)


def _sc_scatter_half(w, dst3, z):
    ep = w.shape[0]
    npad = z.shape[0]
    chunks = dst3.shape[1]
    mesh = plsc.VectorSubcoreMesh(core_axis_name="c", subcore_axis_name="s")

    @functools.partial(
        pl.kernel,
        out_type=jax.ShapeDtypeStruct((_NC, npad, _W), jnp.float32),
        mesh=mesh,
        scratch_types=[
            pltpu.VMEM((chunks, _B), jnp.int32),
            pltpu.VMEM_SHARED((npad, _W), jnp.float32),
            pltpu.VMEM((_B, _W), jnp.float32),
            pltpu.VMEM((_B, _W), jnp.float32),
            pltpu.SemaphoreType.DMA, pltpu.SemaphoreType.DMA,
            pltpu.SemaphoreType.DMA, pltpu.SemaphoreType.DMA,
        ],
    )
    def k(w_hbm, di_hbm, z_hbm, o_hbm, di_v, acc, wv0, wv1, ls0, ls1, ss0, ss1):
        cid = lax.axis_index("c")
        sid = lax.axis_index("s")
        r0 = sid * _NPAD_SUB
        pltpu.sync_copy(z_hbm.at[pl.ds(r0, _NPAD_SUB)],
                        acc.at[pl.ds(r0, _NPAD_SUB)])
        wid = sid * _NC + cid
        row0 = wid * chunks
        pltpu.sync_copy(di_hbm.at[wid], di_v)
        plsc.subcore_barrier()
        wv, lsem, ssem = (wv0, wv1), (ls0, ls1), (ss0, ss1)

        @pl.loop(0, chunks, step=2)
        def _(i0):
            hl = []
            for j in range(2):
                base = (row0 + i0 + j) * _B
                hl.append(pltpu.async_copy(
                    w_hbm.at[pl.ds(base, _B)], wv[j], lsem[j]))
            hs = []
            for j in range(2):
                hl[j].wait()
                hs.append(pltpu.async_copy(
                    wv[j], acc.at[di_v.at[i0 + j]], ssem[j], add=True))
            for hh in hs:
                hh.wait()

        plsc.subcore_barrier()
        pltpu.sync_copy(acc.at[pl.ds(r0, _NPAD_SUB)],
                        o_hbm.at[cid, pl.ds(r0, _NPAD_SUB)])

    return k(w, dst3, z)


def kernel(x, edge_index, W1l, W1r, a1, b1, W2l, W2r, a2, b2, W3l, W3r, a3, b3):
    n = x.shape[0]
    e = edge_index.shape[1]
    npad = _NS * _NPAD_SUB
    loops = jnp.arange(n, dtype=edge_index.dtype)
    src = jnp.concatenate([edge_index[0], loops])
    dst = jnp.concatenate([edge_index[1], loops])
    et = e + n
    grain = _NW * _B * _GP
    ep = ((et + grain - 1) // grain) * grain
    pad = ep - et
    srcp = jnp.concatenate([src, jnp.zeros((pad,), src.dtype)])
    dstp = jnp.concatenate([dst, jnp.full((pad,), npad - 1, dst.dtype)])
    src3 = srcp.reshape(_NW, -1, _B)
    dst3 = dstp.reshape(_NW, -1, _B)
    dstd = dstp.reshape(_NS, 2, -1, _B)
    z = jnp.zeros((npad, _W), jnp.float32)

    xl, xr = _tc_mm2_pair(x, W1l, W1r)
    gs, gd = _sc_gather(xl, xr, src3, dst3)
    w, aw = _tc_edge1(gs, gd, a1, et)
    o1, o2 = _sc_scatter_dual(w, aw, dstd, z)
    h = _tc_finish1(o1, o2, b1, n)

    for wl, wr, att, b, last in ((W2l, W2r, a2, b2, False),
                                 (W3l, W3r, a3, b3, True)):
        hdim = wl.shape[1]
        t = _tc_mm2_combined(h, wl, wr)
        att_p = jnp.zeros((_W,), jnp.float32).at[:hdim].set(att).reshape(8, 16)
        w = _sc_fused23(t, att_p, src3, dst3, hdim)
        p = _sc_scatter_half(w, dst3, z)
        h = _tc_finish23(p, b, n, hdim, last)
    return h
